# speedup vs baseline: 11.8100x; 11.8100x over previous
import functools

import jax
import jax.numpy as jnp
from jax import lax
from jax.experimental import pallas as pl
from jax.experimental.pallas import tpu as pltpu
from jax.experimental.pallas import tpu_sc as plsc

NC = 2
NS = 16
NW = NC * NS
K = 128


def _mesh():
    return plsc.VectorSubcoreMesh(
        core_axis_name="c", subcore_axis_name="s", num_cores=NC, num_subcores=NS
    )


def _make_deg_kernel(npad, ch):
    slab = npad // NS

    def body(dstm, out, idx_d, ones_v, zbuf, deg_sh, sem):
        cid = lax.axis_index("c")
        sid = lax.axis_index("s")
        wid = cid * NS + sid
        for j in range(K // 16):
            ones_v[pl.ds(j * 16, 16)] = jnp.full((16,), 1.0, jnp.float32)
        for j in range(slab // 16):
            zbuf[pl.ds(j * 16, 16)] = jnp.zeros((16,), jnp.float32)
        pltpu.sync_copy(zbuf, deg_sh.at[pl.ds(sid * slab, slab)])
        plsc.subcore_barrier()
        pltpu.sync_copy(dstm.at[wid], idx_d)

        def chunk(i, carry):
            pltpu.sync_copy(ones_v, deg_sh.at[idx_d.at[i]], add=True)
            return carry

        lax.fori_loop(0, ch, chunk, 0)
        plsc.subcore_barrier()
        pltpu.sync_copy(
            deg_sh.at[pl.ds(sid * slab, slab)],
            out.at[cid].at[pl.ds(sid * slab, slab)],
        )

    return pl.kernel(
        body,
        out_type=jax.ShapeDtypeStruct((NC, npad), jnp.float32),
        mesh=_mesh(),
        scratch_types=[
            pltpu.VMEM((ch, K), jnp.int32),
            pltpu.VMEM((K,), jnp.float32),
            pltpu.VMEM((npad // NS,), jnp.float32),
            pltpu.VMEM_SHARED((npad,), jnp.float32),
            pltpu.SemaphoreType.DMA,
        ],
    )


def _make_edge_kernel(npad, d, ch):
    slab = npad // NS

    def body(hs_hbm, srcm, dstm, zeros_hbm, out, idx_s, idx_d, rows, acc, sem):
        cid = lax.axis_index("c")
        sid = lax.axis_index("s")
        wid = cid * NS + sid
        pltpu.sync_copy(
            zeros_hbm.at[pl.ds(sid * slab, slab)],
            acc.at[pl.ds(sid * slab, slab)],
        )
        pltpu.sync_copy(srcm.at[wid], idx_s)
        pltpu.sync_copy(dstm.at[wid], idx_d)
        plsc.subcore_barrier()

        def chunk(i, carry):
            pltpu.async_copy(hs_hbm.at[idx_s.at[i]], rows, sem).wait()
            pltpu.sync_copy(rows, acc.at[idx_d.at[i]], add=True)
            return carry

        lax.fori_loop(0, ch, chunk, 0)
        plsc.subcore_barrier()
        pltpu.sync_copy(
            acc.at[pl.ds(sid * slab, slab)],
            out.at[cid].at[pl.ds(sid * slab, slab)],
        )

    return pl.kernel(
        body,
        out_type=jax.ShapeDtypeStruct((NC, npad, d), jnp.float32),
        mesh=_mesh(),
        scratch_types=[
            pltpu.VMEM((ch, K), jnp.int32),
            pltpu.VMEM((ch, K), jnp.int32),
            pltpu.VMEM((K, d), jnp.float32),
            pltpu.VMEM_SHARED((npad, d), jnp.float32),
            pltpu.SemaphoreType.DMA,
        ],
    )


def _mm_scale_body(deg_ref, x_ref, w_ref, hs_ref):
    dinv = lax.rsqrt(deg_ref[...] + 1.0)
    h = jnp.dot(x_ref[...], w_ref[...], preferred_element_type=jnp.float32)
    hs_ref[...] = h * dinv


def _mid_body(acc_ref, hs_ref, deg_ref, b_ref, w_ref, e_ref, hs2_ref):
    dinv = lax.rsqrt(deg_ref[...] + 1.0)
    t = (acc_ref[0] + acc_ref[1] + hs_ref[...]) * dinv + b_ref[...]
    e = jnp.maximum(t, 0.0)
    e_ref[...] = e
    h2 = jnp.dot(e, w_ref[...], preferred_element_type=jnp.float32)
    hs2_ref[...] = h2 * dinv


def _final_body(acc_ref, hs_ref, deg_ref, b_ref, out_ref):
    dinv = lax.rsqrt(deg_ref[...] + 1.0)
    out_ref[...] = (acc_ref[0] + acc_ref[1] + hs_ref[...]) * dinv + b_ref[...]


def kernel(x, edge_index, W0, b0, W1, b1):
    n, d = x.shape
    e = edge_index.shape[1]
    npad = ((n + 1 + 1023) // 1024) * 1024
    ew = ((e + NW * K - 1) // (NW * K)) * K
    ch = ew // K
    epad = NW * ew

    xp = jnp.pad(x, ((0, npad - n), (0, 0)))
    pad_idx = jnp.full((epad - e,), n, dtype=edge_index.dtype)
    srcm = jnp.concatenate([edge_index[0], pad_idx]).reshape(NW, ch, K)
    dstm = jnp.concatenate([edge_index[1], pad_idx]).reshape(NW, ch, K)
    zeros2d = jnp.zeros((npad, d), jnp.float32)
    b0r = b0.reshape(1, d)
    b1r = b1.reshape(1, d)

    deg_part = _make_deg_kernel(npad, ch)(dstm)
    deg_col = (deg_part[0] + deg_part[1]).reshape(npad, 1)

    br = 1024
    grid = (npad // br,)
    mm_scale = pl.pallas_call(
        _mm_scale_body,
        grid=grid,
        in_specs=[
            pl.BlockSpec((br, 1), lambda i: (i, 0)),
            pl.BlockSpec((br, d), lambda i: (i, 0)),
            pl.BlockSpec((d, d), lambda i: (0, 0)),
        ],
        out_specs=pl.BlockSpec((br, d), lambda i: (i, 0)),
        out_shape=jax.ShapeDtypeStruct((npad, d), jnp.float32),
    )
    hs1 = mm_scale(deg_col, xp, W0)

    edge_pass = _make_edge_kernel(npad, d, ch)
    acc1 = edge_pass(hs1, srcm, dstm, zeros2d)

    mid = pl.pallas_call(
        _mid_body,
        grid=grid,
        in_specs=[
            pl.BlockSpec((NC, br, d), lambda i: (0, i, 0)),
            pl.BlockSpec((br, d), lambda i: (i, 0)),
            pl.BlockSpec((br, 1), lambda i: (i, 0)),
            pl.BlockSpec((1, d), lambda i: (0, 0)),
            pl.BlockSpec((d, d), lambda i: (0, 0)),
        ],
        out_specs=[
            pl.BlockSpec((br, d), lambda i: (i, 0)),
            pl.BlockSpec((br, d), lambda i: (i, 0)),
        ],
        out_shape=[
            jax.ShapeDtypeStruct((npad, d), jnp.float32),
            jax.ShapeDtypeStruct((npad, d), jnp.float32),
        ],
    )
    e1, hs2 = mid(acc1, hs1, deg_col, b0r, W1)

    acc2 = edge_pass(hs2, srcm, dstm, zeros2d)

    final = pl.pallas_call(
        _final_body,
        grid=grid,
        in_specs=[
            pl.BlockSpec((NC, br, d), lambda i: (0, i, 0)),
            pl.BlockSpec((br, d), lambda i: (i, 0)),
            pl.BlockSpec((br, 1), lambda i: (i, 0)),
            pl.BlockSpec((1, d), lambda i: (0, 0)),
        ],
        out_specs=pl.BlockSpec((br, d), lambda i: (i, 0)),
        out_shape=jax.ShapeDtypeStruct((npad, d), jnp.float32),
    )
    out2 = final(acc2, hs2, deg_col, b1r)

    return (x, e1[:n], out2[:n])


# speedup vs baseline: 14.7203x; 1.2464x over previous
import functools

import jax
import jax.numpy as jnp
from jax import lax
from jax.experimental import pallas as pl
from jax.experimental.pallas import tpu as pltpu
from jax.experimental.pallas import tpu_sc as plsc

NC = 2
NS = 16
NW = NC * NS
K = 128
R = 3


def _mesh():
    return plsc.VectorSubcoreMesh(
        core_axis_name="c", subcore_axis_name="s", num_cores=NC, num_subcores=NS
    )


def _make_deg_kernel(npad, ch):
    slab = npad // NS

    def body(pair, out, idxp, ones_v, zbuf, deg_sh, sem):
        cid = lax.axis_index("c")
        sid = lax.axis_index("s")
        wid = cid * NS + sid
        for j in range(K // 16):
            ones_v[pl.ds(j * 16, 16)] = jnp.full((16,), 1.0, jnp.float32)
        for j in range(slab // 16):
            zbuf[pl.ds(j * 16, 16)] = jnp.zeros((16,), jnp.float32)
        pltpu.sync_copy(zbuf, deg_sh.at[pl.ds(sid * slab, slab)])
        plsc.subcore_barrier()
        pltpu.sync_copy(pair.at[wid], idxp)

        def chunk(i, carry):
            pltpu.sync_copy(ones_v, deg_sh.at[idxp.at[i, 1]], add=True)
            return carry

        lax.fori_loop(0, ch, chunk, 0)
        plsc.subcore_barrier()
        pltpu.sync_copy(
            deg_sh.at[pl.ds(sid * slab, slab)],
            out.at[cid].at[pl.ds(sid * slab, slab)],
        )

    return pl.kernel(
        body,
        out_type=jax.ShapeDtypeStruct((NC, npad), jnp.float32),
        mesh=_mesh(),
        scratch_types=[
            pltpu.VMEM((ch, 2, K), jnp.int32),
            pltpu.VMEM((K,), jnp.float32),
            pltpu.VMEM((npad // NS,), jnp.float32),
            pltpu.VMEM_SHARED((npad,), jnp.float32),
            pltpu.SemaphoreType.DMA,
        ],
    )


def _make_edge_kernel(npad, d, ch):
    slab = npad // NS

    def body(hs_hbm, pair, zeros_hbm, out, idxb, rows, acc, sem_i, sem_g):
        cid = lax.axis_index("c")
        sid = lax.axis_index("s")
        wid = cid * NS + sid
        pltpu.sync_copy(
            zeros_hbm.at[pl.ds(sid * slab, slab)],
            acc.at[pl.ds(sid * slab, slab)],
        )
        for b in range(R):
            pltpu.async_copy(pair.at[wid, b], idxb.at[b], sem_i.at[b])
        for b in range(R - 1):
            pltpu.make_async_copy(
                pair.at[wid, b], idxb.at[b], sem_i.at[b]
            ).wait()
            pltpu.async_copy(
                hs_hbm.at[idxb.at[b, 0]], rows.at[b], sem_g.at[b]
            )
        plsc.subcore_barrier()

        def chunk(i, carry):
            b = lax.rem(i, R)
            bp = lax.rem(i + R - 1, R)
            j = i + R - 1

            @pl.when(j < ch)
            def _():
                pltpu.make_async_copy(
                    pair.at[wid, j], idxb.at[bp], sem_i.at[bp]
                ).wait()
                pltpu.async_copy(
                    hs_hbm.at[idxb.at[bp, 0]], rows.at[bp], sem_g.at[bp]
                )

            pltpu.make_async_copy(
                hs_hbm.at[idxb.at[b, 0]], rows.at[b], sem_g.at[b]
            ).wait()
            pltpu.sync_copy(rows.at[b], acc.at[idxb.at[b, 1]], add=True)

            @pl.when(i + R < ch)
            def _():
                pltpu.async_copy(pair.at[wid, i + R], idxb.at[b], sem_i.at[b])

            return carry

        lax.fori_loop(0, ch, chunk, 0)
        plsc.subcore_barrier()
        pltpu.sync_copy(
            acc.at[pl.ds(sid * slab, slab)],
            out.at[cid].at[pl.ds(sid * slab, slab)],
        )

    return pl.kernel(
        body,
        out_type=jax.ShapeDtypeStruct((NC, npad, d), jnp.float32),
        mesh=_mesh(),
        scratch_types=[
            pltpu.VMEM((R, 2, K), jnp.int32),
            pltpu.VMEM((R, K, d), jnp.float32),
            pltpu.VMEM_SHARED((npad, d), jnp.float32),
            pltpu.SemaphoreType.DMA((R,)),
            pltpu.SemaphoreType.DMA((R,)),
        ],
    )


def _mm_scale_body(deg_ref, x_ref, w_ref, hs_ref):
    dinv = lax.rsqrt(deg_ref[...] + 1.0)
    h = jnp.dot(x_ref[...], w_ref[...], preferred_element_type=jnp.float32)
    hs_ref[...] = h * dinv


def _mid_body(acc_ref, hs_ref, deg_ref, b_ref, w_ref, e_ref, hs2_ref):
    dinv = lax.rsqrt(deg_ref[...] + 1.0)
    t = (acc_ref[0] + acc_ref[1] + hs_ref[...]) * dinv + b_ref[...]
    e = jnp.maximum(t, 0.0)
    e_ref[...] = e
    h2 = jnp.dot(e, w_ref[...], preferred_element_type=jnp.float32)
    hs2_ref[...] = h2 * dinv


def _final_body(acc_ref, hs_ref, deg_ref, b_ref, out_ref):
    dinv = lax.rsqrt(deg_ref[...] + 1.0)
    out_ref[...] = (acc_ref[0] + acc_ref[1] + hs_ref[...]) * dinv + b_ref[...]


def kernel(x, edge_index, W0, b0, W1, b1):
    n, d = x.shape
    e = edge_index.shape[1]
    npad = ((n + 1 + 127) // 128) * 128
    npad_d = ((n + 1 + 255) // 256) * 256
    ew = ((e + NW * K - 1) // (NW * K)) * K
    ch = ew // K
    epad = NW * ew

    xp = jnp.pad(x, ((0, npad - n), (0, 0)))
    pad_idx = jnp.full((epad - e,), n, dtype=edge_index.dtype)
    srcm = jnp.concatenate([edge_index[0], pad_idx]).reshape(NW, ch, K)
    dstm = jnp.concatenate([edge_index[1], pad_idx]).reshape(NW, ch, K)
    pair = jnp.stack([srcm, dstm], axis=2)
    zeros2d = jnp.zeros((npad, d), jnp.float32)
    b0r = b0.reshape(1, d)
    b1r = b1.reshape(1, d)

    deg_part = _make_deg_kernel(npad_d, ch)(pair)
    deg_col = (deg_part[0] + deg_part[1])[:npad].reshape(npad, 1)

    br = npad // 8
    grid = (npad // br,)
    mm_scale = pl.pallas_call(
        _mm_scale_body,
        grid=grid,
        in_specs=[
            pl.BlockSpec((br, 1), lambda i: (i, 0)),
            pl.BlockSpec((br, d), lambda i: (i, 0)),
            pl.BlockSpec((d, d), lambda i: (0, 0)),
        ],
        out_specs=pl.BlockSpec((br, d), lambda i: (i, 0)),
        out_shape=jax.ShapeDtypeStruct((npad, d), jnp.float32),
    )
    hs1 = mm_scale(deg_col, xp, W0)

    edge_pass = _make_edge_kernel(npad, d, ch)
    acc1 = edge_pass(hs1, pair, zeros2d)

    mid = pl.pallas_call(
        _mid_body,
        grid=grid,
        in_specs=[
            pl.BlockSpec((NC, br, d), lambda i: (0, i, 0)),
            pl.BlockSpec((br, d), lambda i: (i, 0)),
            pl.BlockSpec((br, 1), lambda i: (i, 0)),
            pl.BlockSpec((1, d), lambda i: (0, 0)),
            pl.BlockSpec((d, d), lambda i: (0, 0)),
        ],
        out_specs=[
            pl.BlockSpec((br, d), lambda i: (i, 0)),
            pl.BlockSpec((br, d), lambda i: (i, 0)),
        ],
        out_shape=[
            jax.ShapeDtypeStruct((npad, d), jnp.float32),
            jax.ShapeDtypeStruct((npad, d), jnp.float32),
        ],
    )
    e1, hs2 = mid(acc1, hs1, deg_col, b0r, W1)

    acc2 = edge_pass(hs2, pair, zeros2d)

    final = pl.pallas_call(
        _final_body,
        grid=grid,
        in_specs=[
            pl.BlockSpec((NC, br, d), lambda i: (0, i, 0)),
            pl.BlockSpec((br, d), lambda i: (i, 0)),
            pl.BlockSpec((br, 1), lambda i: (i, 0)),
            pl.BlockSpec((1, d), lambda i: (0, 0)),
        ],
        out_specs=pl.BlockSpec((br, d), lambda i: (i, 0)),
        out_shape=jax.ShapeDtypeStruct((npad, d), jnp.float32),
    )
    out2 = final(acc2, hs2, deg_col, b1r)

    return (x, e1[:n], out2[:n])


# speedup vs baseline: 21.5040x; 1.4608x over previous
import functools

import jax
import jax.numpy as jnp
from jax import lax
from jax.experimental import pallas as pl
from jax.experimental.pallas import tpu as pltpu
from jax.experimental.pallas import tpu_sc as plsc

NC = 2
NS = 16
NW = NC * NS
K = 128
R = 3
F0 = 0.71


def _mesh():
    return plsc.VectorSubcoreMesh(
        core_axis_name="c", subcore_axis_name="s", num_cores=NC, num_subcores=NS
    )


def _make_deg_kernel(npad, ch0, ch1):
    slab = npad // NS

    def body(pair, out, idxp, ones_v, zbuf, deg_sh, sem):
        cid = lax.axis_index("c")
        sid = lax.axis_index("s")
        wid = cid * NS + sid
        nch = jnp.where(cid == 0, ch0, ch1)
        for j in range(K // 16):
            ones_v[pl.ds(j * 16, 16)] = jnp.full((16,), 1.0, jnp.float32)
        for j in range(slab // 16):
            zbuf[pl.ds(j * 16, 16)] = jnp.zeros((16,), jnp.float32)
        pltpu.sync_copy(zbuf, deg_sh.at[pl.ds(sid * slab, slab)])
        plsc.subcore_barrier()
        pltpu.sync_copy(pair.at[wid], idxp)

        def chunk(i, carry):
            pltpu.sync_copy(ones_v, deg_sh.at[idxp.at[i, 1]], add=True)
            return carry

        lax.fori_loop(0, nch, chunk, 0)
        plsc.subcore_barrier()
        pltpu.sync_copy(
            deg_sh.at[pl.ds(sid * slab, slab)],
            out.at[cid].at[pl.ds(sid * slab, slab)],
        )

    return pl.kernel(
        body,
        out_type=jax.ShapeDtypeStruct((NC, npad), jnp.float32),
        mesh=_mesh(),
        scratch_types=[
            pltpu.VMEM((max(ch0, ch1), 2, K), jnp.int32),
            pltpu.VMEM((K,), jnp.float32),
            pltpu.VMEM((npad // NS,), jnp.float32),
            pltpu.VMEM_SHARED((npad,), jnp.float32),
            pltpu.SemaphoreType.DMA,
        ],
    )


def _make_edge_kernel(npad, d, ch0, ch1):
    slab = npad // NS

    def body(hs_hbm, pair, zeros_hbm, out, idxb, rows, acc, sem_i, sem_g):
        cid = lax.axis_index("c")
        sid = lax.axis_index("s")
        wid = cid * NS + sid
        nch = jnp.where(cid == 0, ch0, ch1)
        pltpu.sync_copy(
            zeros_hbm.at[pl.ds(sid * slab, slab)],
            acc.at[pl.ds(sid * slab, slab)],
        )
        for b in range(R):
            pltpu.async_copy(pair.at[wid, b], idxb.at[b], sem_i.at[b])
        for b in range(R - 1):
            pltpu.make_async_copy(
                pair.at[wid, b], idxb.at[b], sem_i.at[b]
            ).wait()
            pltpu.async_copy(
                hs_hbm.at[idxb.at[b, 0]], rows.at[b], sem_g.at[b]
            )
        plsc.subcore_barrier()

        def chunk(i, carry):
            b = lax.rem(i, R)
            bp = lax.rem(i + R - 1, R)
            j = i + R - 1

            @pl.when(j < nch)
            def _():
                pltpu.make_async_copy(
                    pair.at[wid, j], idxb.at[bp], sem_i.at[bp]
                ).wait()
                pltpu.async_copy(
                    hs_hbm.at[idxb.at[bp, 0]], rows.at[bp], sem_g.at[bp]
                )

            pltpu.make_async_copy(
                hs_hbm.at[idxb.at[b, 0]], rows.at[b], sem_g.at[b]
            ).wait()
            pltpu.sync_copy(rows.at[b], acc.at[idxb.at[b, 1]], add=True)

            @pl.when(i + R < nch)
            def _():
                pltpu.async_copy(pair.at[wid, i + R], idxb.at[b], sem_i.at[b])

            return carry

        lax.fori_loop(0, nch, chunk, 0)
        plsc.subcore_barrier()
        pltpu.sync_copy(
            acc.at[pl.ds(sid * slab, slab)],
            out.at[cid].at[pl.ds(sid * slab, slab)],
        )

    return pl.kernel(
        body,
        out_type=jax.ShapeDtypeStruct((NC, npad, d), jnp.float32),
        mesh=_mesh(),
        scratch_types=[
            pltpu.VMEM((R, 2, K), jnp.int32),
            pltpu.VMEM((R, K, d), jnp.float32),
            pltpu.VMEM_SHARED((npad, d), jnp.float32),
            pltpu.SemaphoreType.DMA((R,)),
            pltpu.SemaphoreType.DMA((R,)),
        ],
    )


def _mm_scale_body(deg_ref, x_ref, w_ref, hs_ref):
    dinv = lax.rsqrt(deg_ref[...] + 1.0)
    h = jnp.dot(x_ref[...], w_ref[...], preferred_element_type=jnp.float32)
    hs_ref[...] = h * dinv


def _mid_body(acc_ref, hs_ref, deg_ref, b_ref, w_ref, e_ref, hs2_ref):
    dinv = lax.rsqrt(deg_ref[...] + 1.0)
    t = (acc_ref[0] + acc_ref[1] + hs_ref[...]) * dinv + b_ref[...]
    e = jnp.maximum(t, 0.0)
    e_ref[...] = e
    h2 = jnp.dot(e, w_ref[...], preferred_element_type=jnp.float32)
    hs2_ref[...] = h2 * dinv


def _final_body(acc_ref, hs_ref, deg_ref, b_ref, out_ref):
    dinv = lax.rsqrt(deg_ref[...] + 1.0)
    out_ref[...] = (acc_ref[0] + acc_ref[1] + hs_ref[...]) * dinv + b_ref[...]


def kernel(x, edge_index, W0, b0, W1, b1):
    n, d = x.shape
    e = edge_index.shape[1]
    npad = ((n + 1 + 127) // 128) * 128
    npad_d = ((n + 1 + 255) // 256) * 256
    ct = (e + K - 1) // K
    ch0 = max(R, int(round(ct * F0 / NS)))
    ch1 = max(R, -((NS * ch0 - ct) // NS))
    ch_max = max(ch0, ch1)
    epad = NS * (ch0 + ch1) * K

    xp = jnp.pad(x, ((0, npad - n), (0, 0)))
    pad_idx = jnp.full((epad - e,), n, dtype=edge_index.dtype)
    split = NS * ch0 * K

    def _worker_chunks(flat):
        a = flat[:split].reshape(NS, ch0, K)
        bpart = flat[split:].reshape(NS, ch1, K)
        a = jnp.pad(a, ((0, 0), (0, ch_max - ch0), (0, 0)), constant_values=n)
        bpart = jnp.pad(
            bpart, ((0, 0), (0, ch_max - ch1), (0, 0)), constant_values=n
        )
        return jnp.concatenate([a, bpart], axis=0)

    srcm = _worker_chunks(jnp.concatenate([edge_index[0], pad_idx]))
    dstm = _worker_chunks(jnp.concatenate([edge_index[1], pad_idx]))
    pair = jnp.stack([srcm, dstm], axis=2)
    zeros2d = jnp.zeros((npad, d), jnp.float32)
    b0r = b0.reshape(1, d)
    b1r = b1.reshape(1, d)

    deg_part = _make_deg_kernel(npad_d, ch0, ch1)(pair)
    deg_col = (deg_part[0] + deg_part[1])[:npad].reshape(npad, 1)

    br = npad // 8
    grid = (npad // br,)
    mm_scale = pl.pallas_call(
        _mm_scale_body,
        grid=grid,
        in_specs=[
            pl.BlockSpec((br, 1), lambda i: (i, 0)),
            pl.BlockSpec((br, d), lambda i: (i, 0)),
            pl.BlockSpec((d, d), lambda i: (0, 0)),
        ],
        out_specs=pl.BlockSpec((br, d), lambda i: (i, 0)),
        out_shape=jax.ShapeDtypeStruct((npad, d), jnp.float32),
    )
    hs1 = mm_scale(deg_col, xp, W0)

    edge_pass = _make_edge_kernel(npad, d, ch0, ch1)
    acc1 = edge_pass(hs1, pair, zeros2d)

    mid = pl.pallas_call(
        _mid_body,
        grid=grid,
        in_specs=[
            pl.BlockSpec((NC, br, d), lambda i: (0, i, 0)),
            pl.BlockSpec((br, d), lambda i: (i, 0)),
            pl.BlockSpec((br, 1), lambda i: (i, 0)),
            pl.BlockSpec((1, d), lambda i: (0, 0)),
            pl.BlockSpec((d, d), lambda i: (0, 0)),
        ],
        out_specs=[
            pl.BlockSpec((br, d), lambda i: (i, 0)),
            pl.BlockSpec((br, d), lambda i: (i, 0)),
        ],
        out_shape=[
            jax.ShapeDtypeStruct((npad, d), jnp.float32),
            jax.ShapeDtypeStruct((npad, d), jnp.float32),
        ],
    )
    e1, hs2 = mid(acc1, hs1, deg_col, b0r, W1)

    acc2 = edge_pass(hs2, pair, zeros2d)

    final = pl.pallas_call(
        _final_body,
        grid=grid,
        in_specs=[
            pl.BlockSpec((NC, br, d), lambda i: (0, i, 0)),
            pl.BlockSpec((br, d), lambda i: (i, 0)),
            pl.BlockSpec((br, 1), lambda i: (i, 0)),
            pl.BlockSpec((1, d), lambda i: (0, 0)),
        ],
        out_specs=pl.BlockSpec((br, d), lambda i: (i, 0)),
        out_shape=jax.ShapeDtypeStruct((npad, d), jnp.float32),
    )
    out2 = final(acc2, hs2, deg_col, b1r)

    return (x, e1[:n], out2[:n])


# speedup vs baseline: 23.2529x; 1.0813x over previous
import functools

import jax
import jax.numpy as jnp
from jax import lax
from jax.experimental import pallas as pl
from jax.experimental.pallas import tpu as pltpu
from jax.experimental.pallas import tpu_sc as plsc

NC = 2
NS = 16
NW = NC * NS
K = 128
R = 3
F0 = 0.693


def _mesh():
    return plsc.VectorSubcoreMesh(
        core_axis_name="c", subcore_axis_name="s", num_cores=NC, num_subcores=NS
    )


def _chunk_range(ct):
    s0 = min(ct, max(0, int(round(ct * F0))))
    s1 = ct - s0
    return s0, s0 // NS, s0 % NS, s1 // NS, s1 % NS


def _worker_span(cid, sid, s0, q0, r0, q1, r1):
    nch = jnp.where(cid == 0, q0 + (sid < r0), q1 + (sid < r1))
    base = jnp.where(
        cid == 0,
        sid * q0 + jnp.minimum(sid, r0),
        s0 + sid * q1 + jnp.minimum(sid, r1),
    )
    return nch.astype(jnp.int32), base.astype(jnp.int32)


def _make_deg_kernel(nhist, ct):
    slab = nhist // NS
    s0, q0, r0, q1, r1 = _chunk_range(ct)

    def body(er, out, idxd, ones_v, zbuf, deg_sh, sem_i):
        cid = lax.axis_index("c")
        sid = lax.axis_index("s")
        nch, base = _worker_span(cid, sid, s0, q0, r0, q1, r1)
        for j in range(K // 16):
            ones_v[pl.ds(j * 16, 16)] = jnp.full((16,), 1.0, jnp.float32)
        for j in range(slab // 16):
            zbuf[pl.ds(j * 16, 16)] = jnp.zeros((16,), jnp.float32)
        pltpu.sync_copy(zbuf, deg_sh.at[pl.ds(sid * slab, slab)])
        for b in range(R):
            @pl.when(b < nch)
            def _():
                pltpu.async_copy(er.at[1, base + b], idxd.at[b], sem_i.at[b])
        plsc.subcore_barrier()

        def chunk(i, carry):
            b = lax.rem(i, R)
            pltpu.make_async_copy(
                er.at[1, base + i], idxd.at[b], sem_i.at[b]
            ).wait()
            pltpu.sync_copy(ones_v, deg_sh.at[idxd.at[b, 0]], add=True)

            @pl.when(i + R < nch)
            def _():
                pltpu.async_copy(
                    er.at[1, base + i + R], idxd.at[b], sem_i.at[b]
                )

            return carry

        lax.fori_loop(0, nch, chunk, 0)
        plsc.subcore_barrier()
        pltpu.sync_copy(
            deg_sh.at[pl.ds(sid * slab, slab)],
            out.at[cid].at[pl.ds(sid * slab, slab)],
        )

    return pl.kernel(
        body,
        out_type=jax.ShapeDtypeStruct((NC, nhist), jnp.float32),
        mesh=_mesh(),
        scratch_types=[
            pltpu.VMEM((R, 1, K), jnp.int32),
            pltpu.VMEM((K,), jnp.float32),
            pltpu.VMEM((nhist // NS,), jnp.float32),
            pltpu.VMEM_SHARED((nhist,), jnp.float32),
            pltpu.SemaphoreType.DMA((R,)),
        ],
    )


def _make_edge_kernel(npad, n, d, ct):
    slab = npad // NS
    s0, q0, r0, q1, r1 = _chunk_range(ct)

    def body(hs_hbm, er, zeros_hbm, out, idxb, rows, acc, sem_i, sem_g):
        cid = lax.axis_index("c")
        sid = lax.axis_index("s")
        nch, base = _worker_span(cid, sid, s0, q0, r0, q1, r1)
        pltpu.sync_copy(
            zeros_hbm.at[pl.ds(sid * slab, slab)],
            acc.at[pl.ds(sid * slab, slab)],
        )

        def fetch_idx(c, b):
            pltpu.async_copy(er.at[0, c], idxb.at[b, pl.ds(0, 1)], sem_i.at[b])
            pltpu.async_copy(er.at[1, c], idxb.at[b, pl.ds(1, 1)], sem_i.at[b])

        def wait_idx(c, b):
            pltpu.make_async_copy(
                er.at[0, c], idxb.at[b, pl.ds(0, 1)], sem_i.at[b]
            ).wait()
            pltpu.make_async_copy(
                er.at[1, c], idxb.at[b, pl.ds(1, 1)], sem_i.at[b]
            ).wait()

        for b in range(R):
            @pl.when(b < nch)
            def _():
                fetch_idx(base + b, b)
        for b in range(R - 1):
            @pl.when(b < nch)
            def _():
                wait_idx(base + b, b)
                pltpu.async_copy(
                    hs_hbm.at[idxb.at[b, 0]], rows.at[b], sem_g.at[b]
                )
        plsc.subcore_barrier()

        def chunk(i, carry):
            b = lax.rem(i, R)
            bp = lax.rem(i + R - 1, R)
            j = i + R - 1

            @pl.when(j < nch)
            def _():
                wait_idx(base + j, bp)
                pltpu.async_copy(
                    hs_hbm.at[idxb.at[bp, 0]], rows.at[bp], sem_g.at[bp]
                )

            pltpu.make_async_copy(
                hs_hbm.at[idxb.at[b, 0]], rows.at[b], sem_g.at[b]
            ).wait()
            pltpu.sync_copy(rows.at[b], acc.at[idxb.at[b, 1]], add=True)

            @pl.when(i + R < nch)
            def _():
                fetch_idx(base + i + R, b)

            return carry

        lax.fori_loop(0, nch, chunk, 0)
        plsc.subcore_barrier()
        pltpu.sync_copy(
            acc.at[pl.ds(sid * slab, slab)],
            out.at[cid].at[pl.ds(sid * slab, slab)],
        )

    return pl.kernel(
        body,
        out_type=jax.ShapeDtypeStruct((NC, npad, d), jnp.float32),
        mesh=_mesh(),
        scratch_types=[
            pltpu.VMEM((R, 2, K), jnp.int32),
            pltpu.VMEM((R, K, d), jnp.float32),
            pltpu.VMEM_SHARED((npad, d), jnp.float32),
            pltpu.SemaphoreType.DMA((R,)),
            pltpu.SemaphoreType.DMA((R,)),
        ],
    )


def _mm_scale_body(deg_ref, x_ref, w_ref, hs_ref):
    dinv = lax.rsqrt(deg_ref[...] + 1.0)
    h = jnp.dot(x_ref[...], w_ref[...], preferred_element_type=jnp.float32)
    hs_ref[...] = h * dinv


def _mid_body(acc_ref, hs_ref, deg_ref, b_ref, w_ref, e_ref, hs2_ref):
    dinv = lax.rsqrt(deg_ref[...] + 1.0)
    t = (acc_ref[0] + acc_ref[1] + hs_ref[...]) * dinv + b_ref[...]
    e = jnp.maximum(t, 0.0)
    e_ref[...] = e
    h2 = jnp.dot(e, w_ref[...], preferred_element_type=jnp.float32)
    hs2_ref[...] = h2 * dinv


def _final_body(acc_ref, hs_ref, deg_ref, b_ref, out_ref):
    dinv = lax.rsqrt(deg_ref[...] + 1.0)
    out_ref[...] = (acc_ref[0] + acc_ref[1] + hs_ref[...]) * dinv + b_ref[...]


def kernel(x, edge_index, W0, b0, W1, b1):
    n, d = x.shape
    e = edge_index.shape[1]
    assert e % K == 0
    ct = e // K
    npad = ((n + 127) // 128) * 128
    nhist = ((n + 255) // 256) * 256

    er = edge_index.reshape(2, ct, 1, K)
    zeros2d = jnp.zeros((npad, d), jnp.float32)
    b0r = b0.reshape(1, d)
    b1r = b1.reshape(1, d)

    deg_part = _make_deg_kernel(nhist, ct)(er)
    deg_col = (deg_part[0] + deg_part[1])[:n].reshape(n, 1)

    br = npad // 8
    grid = (npad // br,)
    mm_scale = pl.pallas_call(
        _mm_scale_body,
        grid=grid,
        in_specs=[
            pl.BlockSpec((br, 1), lambda i: (i, 0)),
            pl.BlockSpec((br, d), lambda i: (i, 0)),
            pl.BlockSpec((d, d), lambda i: (0, 0)),
        ],
        out_specs=pl.BlockSpec((br, d), lambda i: (i, 0)),
        out_shape=jax.ShapeDtypeStruct((n, d), jnp.float32),
    )
    hs1 = mm_scale(deg_col, x, W0)

    edge_pass = _make_edge_kernel(npad, n, d, ct)
    acc1 = edge_pass(hs1, er, zeros2d)

    mid = pl.pallas_call(
        _mid_body,
        grid=grid,
        in_specs=[
            pl.BlockSpec((NC, br, d), lambda i: (0, i, 0)),
            pl.BlockSpec((br, d), lambda i: (i, 0)),
            pl.BlockSpec((br, 1), lambda i: (i, 0)),
            pl.BlockSpec((1, d), lambda i: (0, 0)),
            pl.BlockSpec((d, d), lambda i: (0, 0)),
        ],
        out_specs=[
            pl.BlockSpec((br, d), lambda i: (i, 0)),
            pl.BlockSpec((br, d), lambda i: (i, 0)),
        ],
        out_shape=[
            jax.ShapeDtypeStruct((n, d), jnp.float32),
            jax.ShapeDtypeStruct((n, d), jnp.float32),
        ],
    )
    e1, hs2 = mid(acc1, hs1, deg_col, b0r, W1)

    acc2 = edge_pass(hs2, er, zeros2d)

    final = pl.pallas_call(
        _final_body,
        grid=grid,
        in_specs=[
            pl.BlockSpec((NC, br, d), lambda i: (0, i, 0)),
            pl.BlockSpec((br, d), lambda i: (i, 0)),
            pl.BlockSpec((br, 1), lambda i: (i, 0)),
            pl.BlockSpec((1, d), lambda i: (0, 0)),
        ],
        out_specs=pl.BlockSpec((br, d), lambda i: (i, 0)),
        out_shape=jax.ShapeDtypeStruct((n, d), jnp.float32),
    )
    out2 = final(acc2, hs2, deg_col, b1r)

    return (x, e1, out2)


# speedup vs baseline: 26.0497x; 1.1203x over previous
import functools

import jax
import jax.numpy as jnp
from jax import lax
from jax.experimental import pallas as pl
from jax.experimental.pallas import tpu as pltpu
from jax.experimental.pallas import tpu_sc as plsc

NC = 2
NS = 16
NW = NC * NS
K = 128
R = 3
RI = 5
F0 = 0.693


def _mesh():
    return plsc.VectorSubcoreMesh(
        core_axis_name="c", subcore_axis_name="s", num_cores=NC, num_subcores=NS
    )


def _chunk_range(ct):
    s0 = min(ct, max(0, int(round(ct * F0))))
    s1 = ct - s0
    return s0, s0 // NS, s0 % NS, s1 // NS, s1 % NS


def _worker_span(cid, sid, s0, q0, r0, q1, r1):
    nch = jnp.where(cid == 0, q0 + (sid < r0), q1 + (sid < r1))
    base = jnp.where(
        cid == 0,
        sid * q0 + jnp.minimum(sid, r0),
        s0 + sid * q1 + jnp.minimum(sid, r1),
    )
    return nch.astype(jnp.int32), base.astype(jnp.int32)


def _slab_copy(src, dst, sid, n):
    slab = ((n // NS) // 8) * 8
    last = n - (NS - 1) * slab

    @pl.when(sid < NS - 1)
    def _():
        pltpu.sync_copy(
            src.at[pl.ds(sid * slab, slab)], dst.at[pl.ds(sid * slab, slab)]
        )

    @pl.when(sid == NS - 1)
    def _():
        pltpu.sync_copy(
            src.at[pl.ds((NS - 1) * slab, last)],
            dst.at[pl.ds((NS - 1) * slab, last)],
        )


def _make_deg_kernel(nhist, ct):
    slab = nhist // NS
    s0, q0, r0, q1, r1 = _chunk_range(ct)

    def body(dstf, out, idxd, ones_v, zbuf, deg_sh, sem_i, sem_s):
        cid = lax.axis_index("c")
        sid = lax.axis_index("s")
        nch, base = _worker_span(cid, sid, s0, q0, r0, q1, r1)
        for j in range(K // 16):
            ones_v[pl.ds(j * 16, 16)] = jnp.full((16,), 1.0, jnp.float32)
        for j in range(slab // 16):
            zbuf[pl.ds(j * 16, 16)] = jnp.zeros((16,), jnp.float32)
        pltpu.sync_copy(zbuf, deg_sh.at[pl.ds(sid * slab, slab)])

        def fetch(c, b):
            pltpu.async_copy(
                dstf.at[pl.ds((base + c) * K, K)], idxd.at[b], sem_i.at[b]
            )

        def wait_fetch(c, b):
            pltpu.make_async_copy(
                dstf.at[pl.ds((base + c) * K, K)], idxd.at[b], sem_i.at[b]
            ).wait()

        def wait_scat(b):
            pltpu.make_async_copy(
                ones_v, deg_sh.at[idxd.at[b]], sem_s.at[b]
            ).wait()

        for b in range(RI):
            @pl.when(b < nch)
            def _():
                fetch(b, b)
        plsc.subcore_barrier()

        def chunk(i, carry):
            bi = lax.rem(i, RI)
            bs = lax.rem(i, 2)
            wait_fetch(i, bi)
            pltpu.async_copy(
                ones_v, deg_sh.at[idxd.at[bi]], sem_s.at[bs], add=True
            )

            @pl.when(i > 0)
            def _():
                wait_scat(lax.rem(i - 1, 2))

                @pl.when(i - 1 + RI < nch)
                def _():
                    fetch(i - 1 + RI, lax.rem(i - 1, RI))

            return carry

        lax.fori_loop(0, nch, chunk, 0)

        @pl.when(nch > 0)
        def _():
            wait_scat(lax.rem(nch - 1, 2))

        plsc.subcore_barrier()
        pltpu.sync_copy(
            deg_sh.at[pl.ds(sid * slab, slab)],
            out.at[cid].at[pl.ds(sid * slab, slab)],
        )

    return pl.kernel(
        body,
        out_type=jax.ShapeDtypeStruct((NC, nhist), jnp.float32),
        mesh=_mesh(),
        scratch_types=[
            pltpu.VMEM((RI, K), jnp.int32),
            pltpu.VMEM((K,), jnp.float32),
            pltpu.VMEM((nhist // NS,), jnp.float32),
            pltpu.VMEM_SHARED((nhist,), jnp.float32),
            pltpu.SemaphoreType.DMA((RI,)),
            pltpu.SemaphoreType.DMA((2,)),
        ],
    )


def _make_edge_kernel(n, d, ct):
    s0, q0, r0, q1, r1 = _chunk_range(ct)

    def body(hs_hbm, srcf, dstf, zeros_hbm, out, idxb, rows, acc,
             sem_i, sem_g, sem_s):
        cid = lax.axis_index("c")
        sid = lax.axis_index("s")
        nch, base = _worker_span(cid, sid, s0, q0, r0, q1, r1)
        _slab_copy(zeros_hbm, acc, sid, n)

        def fetch_idx(c, b):
            pltpu.async_copy(
                srcf.at[pl.ds((base + c) * K, K)], idxb.at[b, 0], sem_i.at[b]
            )
            pltpu.async_copy(
                dstf.at[pl.ds((base + c) * K, K)], idxb.at[b, 1], sem_i.at[b]
            )

        def wait_idx(c, b):
            pltpu.make_async_copy(
                srcf.at[pl.ds((base + c) * K, K)], idxb.at[b, 0], sem_i.at[b]
            ).wait()
            pltpu.make_async_copy(
                dstf.at[pl.ds((base + c) * K, K)], idxb.at[b, 1], sem_i.at[b]
            ).wait()

        def issue_gather(bi, b):
            pltpu.async_copy(
                hs_hbm.at[idxb.at[bi, 0]], rows.at[b], sem_g.at[b]
            )

        def wait_gather(bi, b):
            pltpu.make_async_copy(
                hs_hbm.at[idxb.at[bi, 0]], rows.at[b], sem_g.at[b]
            ).wait()

        def wait_scat(bi, b):
            pltpu.make_async_copy(
                rows.at[b], acc.at[idxb.at[bi, 1]], sem_s.at[b]
            ).wait()

        for b in range(RI):
            @pl.when(b < nch)
            def _():
                fetch_idx(b, b)
        for b in range(R):
            @pl.when(b < nch)
            def _():
                wait_idx(b, b)
                issue_gather(b, b)
        plsc.subcore_barrier()

        def chunk(i, carry):
            b = lax.rem(i, R)
            bi = lax.rem(i, RI)
            wait_gather(bi, b)
            pltpu.async_copy(
                rows.at[b], acc.at[idxb.at[bi, 1]], sem_s.at[b], add=True
            )

            @pl.when(i > 0)
            def _():
                ip = i - 1
                bp = lax.rem(ip, R)
                bip = lax.rem(ip, RI)
                wait_scat(bip, bp)

                @pl.when(ip + RI < nch)
                def _():
                    fetch_idx(ip + RI, bip)

                j = i + R - 1

                @pl.when(j < nch)
                def _():
                    bij = lax.rem(j, RI)
                    wait_idx(j, bij)
                    issue_gather(bij, bp)

            return carry

        lax.fori_loop(0, nch, chunk, 0)

        @pl.when(nch > 0)
        def _():
            wait_scat(lax.rem(nch - 1, RI), lax.rem(nch - 1, R))

        plsc.subcore_barrier()
        _slab_copy(acc, out.at[cid], sid, n)

    return pl.kernel(
        body,
        out_type=jax.ShapeDtypeStruct((NC, n, d), jnp.float32),
        mesh=_mesh(),
        scratch_types=[
            pltpu.VMEM((RI, 2, K), jnp.int32),
            pltpu.VMEM((R, K, d), jnp.float32),
            pltpu.VMEM_SHARED((n, d), jnp.float32),
            pltpu.SemaphoreType.DMA((RI,)),
            pltpu.SemaphoreType.DMA((R,)),
            pltpu.SemaphoreType.DMA((R,)),
        ],
    )


def _mm_scale_body(deg_ref, x_ref, w_ref, hs_ref):
    dinv = lax.rsqrt(deg_ref[...] + 1.0)
    h = jnp.dot(x_ref[...], w_ref[...], preferred_element_type=jnp.float32)
    hs_ref[...] = h * dinv


def _mid_body(acc_ref, hs_ref, deg_ref, b_ref, w_ref, e_ref, hs2_ref):
    dinv = lax.rsqrt(deg_ref[...] + 1.0)
    t = (acc_ref[0] + acc_ref[1] + hs_ref[...]) * dinv + b_ref[...]
    e = jnp.maximum(t, 0.0)
    e_ref[...] = e
    h2 = jnp.dot(e, w_ref[...], preferred_element_type=jnp.float32)
    hs2_ref[...] = h2 * dinv


def _final_body(acc_ref, hs_ref, deg_ref, b_ref, out_ref):
    dinv = lax.rsqrt(deg_ref[...] + 1.0)
    out_ref[...] = (acc_ref[0] + acc_ref[1] + hs_ref[...]) * dinv + b_ref[...]


def kernel(x, edge_index, W0, b0, W1, b1):
    n, d = x.shape
    e = edge_index.shape[1]
    assert e % K == 0 and n % 8 == 0
    ct = e // K
    nhist = ((n + 255) // 256) * 256

    srcf = edge_index[0]
    dstf = edge_index[1]
    zeros2d = jnp.zeros((n, d), jnp.float32)
    b0r = b0.reshape(1, d)
    b1r = b1.reshape(1, d)

    deg_part = _make_deg_kernel(nhist, ct)(dstf)
    deg_col = (deg_part[0] + deg_part[1])[:n].reshape(n, 1)

    br = 1264
    grid = (-(-n // br),)
    mm_scale = pl.pallas_call(
        _mm_scale_body,
        grid=grid,
        in_specs=[
            pl.BlockSpec((br, 1), lambda i: (i, 0)),
            pl.BlockSpec((br, d), lambda i: (i, 0)),
            pl.BlockSpec((d, d), lambda i: (0, 0)),
        ],
        out_specs=pl.BlockSpec((br, d), lambda i: (i, 0)),
        out_shape=jax.ShapeDtypeStruct((n, d), jnp.float32),
    )
    hs1 = mm_scale(deg_col, x, W0)

    edge_pass = _make_edge_kernel(n, d, ct)
    acc1 = edge_pass(hs1, srcf, dstf, zeros2d)

    mid = pl.pallas_call(
        _mid_body,
        grid=grid,
        in_specs=[
            pl.BlockSpec((NC, br, d), lambda i: (0, i, 0)),
            pl.BlockSpec((br, d), lambda i: (i, 0)),
            pl.BlockSpec((br, 1), lambda i: (i, 0)),
            pl.BlockSpec((1, d), lambda i: (0, 0)),
            pl.BlockSpec((d, d), lambda i: (0, 0)),
        ],
        out_specs=[
            pl.BlockSpec((br, d), lambda i: (i, 0)),
            pl.BlockSpec((br, d), lambda i: (i, 0)),
        ],
        out_shape=[
            jax.ShapeDtypeStruct((n, d), jnp.float32),
            jax.ShapeDtypeStruct((n, d), jnp.float32),
        ],
    )
    e1, hs2 = mid(acc1, hs1, deg_col, b0r, W1)

    acc2 = edge_pass(hs2, srcf, dstf, zeros2d)

    final = pl.pallas_call(
        _final_body,
        grid=grid,
        in_specs=[
            pl.BlockSpec((NC, br, d), lambda i: (0, i, 0)),
            pl.BlockSpec((br, d), lambda i: (i, 0)),
            pl.BlockSpec((br, 1), lambda i: (i, 0)),
            pl.BlockSpec((1, d), lambda i: (0, 0)),
        ],
        out_specs=pl.BlockSpec((br, d), lambda i: (i, 0)),
        out_shape=jax.ShapeDtypeStruct((n, d), jnp.float32),
    )
    out2 = final(acc2, hs2, deg_col, b1r)

    return (x, e1, out2)


# speedup vs baseline: 32.2801x; 1.2392x over previous
import functools

import jax
import jax.numpy as jnp
from jax import lax
from jax.experimental import pallas as pl
from jax.experimental.pallas import tpu as pltpu
from jax.experimental.pallas import tpu_sc as plsc

NC = 2
NS = 16
NW = NC * NS
K = 128
R = 3
RI = 5
F0 = 0.536


def _mesh():
    return plsc.VectorSubcoreMesh(
        core_axis_name="c", subcore_axis_name="s", num_cores=NC, num_subcores=NS
    )


def _chunk_range(ct):
    s0 = min(ct, max(0, int(round(ct * F0))))
    s1 = ct - s0
    return s0, s0 // NS, s0 % NS, s1 // NS, s1 % NS


def _worker_span(cid, sid, s0, q0, r0, q1, r1):
    nch = jnp.where(cid == 0, q0 + (sid < r0), q1 + (sid < r1))
    base = jnp.where(
        cid == 0,
        sid * q0 + jnp.minimum(sid, r0),
        s0 + sid * q1 + jnp.minimum(sid, r1),
    )
    return nch.astype(jnp.int32), base.astype(jnp.int32)


def _slab_copy(src, dst, sid, n):
    slab = ((n // NS) // 8) * 8
    last = n - (NS - 1) * slab

    @pl.when(sid < NS - 1)
    def _():
        pltpu.sync_copy(
            src.at[pl.ds(sid * slab, slab)], dst.at[pl.ds(sid * slab, slab)]
        )

    @pl.when(sid == NS - 1)
    def _():
        pltpu.sync_copy(
            src.at[pl.ds((NS - 1) * slab, last)],
            dst.at[pl.ds((NS - 1) * slab, last)],
        )


def _make_deg_kernel(nhist, ct):
    slab = nhist // NS
    s0, q0, r0, q1, r1 = _chunk_range(ct)

    def body(ert, out, idxd, ones_v, zbuf, deg_sh, sem_i, sem_s):
        cid = lax.axis_index("c")
        sid = lax.axis_index("s")
        nch, base = _worker_span(cid, sid, s0, q0, r0, q1, r1)
        for j in range(K // 16):
            ones_v[pl.ds(j * 16, 16)] = jnp.full((16,), 1.0, jnp.float32)
        for j in range(slab // 16):
            zbuf[pl.ds(j * 16, 16)] = jnp.zeros((16,), jnp.float32)
        pltpu.sync_copy(zbuf, deg_sh.at[pl.ds(sid * slab, slab)])

        def fetch(c, b):
            pltpu.async_copy(ert.at[base + c, 1], idxd.at[b], sem_i.at[b])

        def wait_fetch(c, b):
            pltpu.make_async_copy(
                ert.at[base + c, 1], idxd.at[b], sem_i.at[b]
            ).wait()

        def wait_scat(b):
            pltpu.make_async_copy(
                ones_v, deg_sh.at[idxd.at[b, 0]], sem_s.at[b]
            ).wait()

        for b in range(RI):
            @pl.when(b < nch)
            def _():
                fetch(b, b)
        plsc.subcore_barrier()

        def chunk(i, carry):
            bi = lax.rem(i, RI)
            bs = lax.rem(i, 2)
            wait_fetch(i, bi)
            pltpu.async_copy(
                ones_v, deg_sh.at[idxd.at[bi, 0]], sem_s.at[bs], add=True
            )

            @pl.when(i > 0)
            def _():
                wait_scat(lax.rem(i - 1, 2))

                @pl.when(i - 1 + RI < nch)
                def _():
                    fetch(i - 1 + RI, lax.rem(i - 1, RI))

            return carry

        lax.fori_loop(0, nch, chunk, 0)

        @pl.when(nch > 0)
        def _():
            wait_scat(lax.rem(nch - 1, 2))

        plsc.subcore_barrier()
        pltpu.sync_copy(
            deg_sh.at[pl.ds(sid * slab, slab)],
            out.at[cid].at[pl.ds(sid * slab, slab)],
        )

    return pl.kernel(
        body,
        out_type=jax.ShapeDtypeStruct((NC, nhist), jnp.float32),
        mesh=_mesh(),
        scratch_types=[
            pltpu.VMEM((RI, 1, K), jnp.int32),
            pltpu.VMEM((K,), jnp.float32),
            pltpu.VMEM((nhist // NS,), jnp.float32),
            pltpu.VMEM_SHARED((nhist,), jnp.float32),
            pltpu.SemaphoreType.DMA((RI,)),
            pltpu.SemaphoreType.DMA((2,)),
        ],
    )


def _make_edge_kernel(n, d, ct):
    s0, q0, r0, q1, r1 = _chunk_range(ct)

    def body(hs_hbm, ert, zeros_hbm, out, idxb, rows, acc,
             sem_i, sem_g, sem_s):
        cid = lax.axis_index("c")
        sid = lax.axis_index("s")
        nch, base = _worker_span(cid, sid, s0, q0, r0, q1, r1)
        _slab_copy(zeros_hbm, acc, sid, n)

        def fetch_idx(c, b):
            pltpu.async_copy(
                ert.at[base + c, 0], idxb.at[b, pl.ds(0, 1)], sem_i.at[b]
            )
            pltpu.async_copy(
                ert.at[base + c, 1], idxb.at[b, pl.ds(1, 1)], sem_i.at[b]
            )

        def wait_idx(c, b):
            pltpu.make_async_copy(
                ert.at[base + c, 0], idxb.at[b, pl.ds(0, 1)], sem_i.at[b]
            ).wait()
            pltpu.make_async_copy(
                ert.at[base + c, 1], idxb.at[b, pl.ds(1, 1)], sem_i.at[b]
            ).wait()

        def issue_gather(bi, b):
            pltpu.async_copy(
                hs_hbm.at[idxb.at[bi, 0]], rows.at[b], sem_g.at[b]
            )

        def wait_gather(bi, b):
            pltpu.make_async_copy(
                hs_hbm.at[idxb.at[bi, 0]], rows.at[b], sem_g.at[b]
            ).wait()

        def wait_scat(bi, b):
            pltpu.make_async_copy(
                rows.at[b], acc.at[idxb.at[bi, 1]], sem_s.at[b]
            ).wait()

        for b in range(RI):
            @pl.when(b < nch)
            def _():
                fetch_idx(b, b)
        for b in range(R):
            @pl.when(b < nch)
            def _():
                wait_idx(b, b)
                issue_gather(b, b)
        plsc.subcore_barrier()

        def chunk(i, carry):
            b = lax.rem(i, R)
            bi = lax.rem(i, RI)
            wait_gather(bi, b)
            pltpu.async_copy(
                rows.at[b], acc.at[idxb.at[bi, 1]], sem_s.at[b], add=True
            )

            @pl.when(i > 0)
            def _():
                ip = i - 1
                bp = lax.rem(ip, R)
                bip = lax.rem(ip, RI)
                wait_scat(bip, bp)

                @pl.when(ip + RI < nch)
                def _():
                    fetch_idx(ip + RI, bip)

                j = i + R - 1

                @pl.when(j < nch)
                def _():
                    bij = lax.rem(j, RI)
                    wait_idx(j, bij)
                    issue_gather(bij, bp)

            return carry

        lax.fori_loop(0, nch, chunk, 0)

        @pl.when(nch > 0)
        def _():
            wait_scat(lax.rem(nch - 1, RI), lax.rem(nch - 1, R))

        plsc.subcore_barrier()
        _slab_copy(acc, out.at[cid], sid, n)

    return pl.kernel(
        body,
        out_type=jax.ShapeDtypeStruct((NC, n, d), jnp.float32),
        mesh=_mesh(),
        scratch_types=[
            pltpu.VMEM((RI, 2, K), jnp.int32),
            pltpu.VMEM((R, K, d), jnp.float32),
            pltpu.VMEM_SHARED((n, d), jnp.float32),
            pltpu.SemaphoreType.DMA((RI,)),
            pltpu.SemaphoreType.DMA((R,)),
            pltpu.SemaphoreType.DMA((R,)),
        ],
    )


def _mm_scale_body(deg_ref, x_ref, w_ref, hs_ref):
    dinv = lax.rsqrt(deg_ref[...] + 1.0)
    h = jnp.dot(x_ref[...], w_ref[...], preferred_element_type=jnp.float32)
    hs_ref[...] = h * dinv


def _mid_body(acc_ref, hs_ref, deg_ref, b_ref, w_ref, e_ref, hs2_ref):
    dinv = lax.rsqrt(deg_ref[...] + 1.0)
    t = (acc_ref[0] + acc_ref[1] + hs_ref[...]) * dinv + b_ref[...]
    e = jnp.maximum(t, 0.0)
    e_ref[...] = e
    h2 = jnp.dot(e, w_ref[...], preferred_element_type=jnp.float32)
    hs2_ref[...] = h2 * dinv


def _final_body(acc_ref, hs_ref, deg_ref, b_ref, out_ref):
    dinv = lax.rsqrt(deg_ref[...] + 1.0)
    out_ref[...] = (acc_ref[0] + acc_ref[1] + hs_ref[...]) * dinv + b_ref[...]


def kernel(x, edge_index, W0, b0, W1, b1):
    n, d = x.shape
    e = edge_index.shape[1]
    assert e % K == 0 and n % 8 == 0
    ct = e // K
    nhist = ((n + 255) // 256) * 256

    ert = jnp.transpose(edge_index.reshape(2, ct, 1, K), (1, 0, 2, 3))
    zeros2d = jnp.zeros((n, d), jnp.float32)
    b0r = b0.reshape(1, d)
    b1r = b1.reshape(1, d)

    deg_part = _make_deg_kernel(nhist, ct)(ert)
    deg_col = (deg_part[0] + deg_part[1])[:n].reshape(n, 1)

    br = 1264
    grid = (-(-n // br),)
    mm_scale = pl.pallas_call(
        _mm_scale_body,
        grid=grid,
        in_specs=[
            pl.BlockSpec((br, 1), lambda i: (i, 0)),
            pl.BlockSpec((br, d), lambda i: (i, 0)),
            pl.BlockSpec((d, d), lambda i: (0, 0)),
        ],
        out_specs=pl.BlockSpec((br, d), lambda i: (i, 0)),
        out_shape=jax.ShapeDtypeStruct((n, d), jnp.float32),
    )
    hs1 = mm_scale(deg_col, x, W0)

    edge_pass = _make_edge_kernel(n, d, ct)
    acc1 = edge_pass(hs1, ert, zeros2d)

    mid = pl.pallas_call(
        _mid_body,
        grid=grid,
        in_specs=[
            pl.BlockSpec((NC, br, d), lambda i: (0, i, 0)),
            pl.BlockSpec((br, d), lambda i: (i, 0)),
            pl.BlockSpec((br, 1), lambda i: (i, 0)),
            pl.BlockSpec((1, d), lambda i: (0, 0)),
            pl.BlockSpec((d, d), lambda i: (0, 0)),
        ],
        out_specs=[
            pl.BlockSpec((br, d), lambda i: (i, 0)),
            pl.BlockSpec((br, d), lambda i: (i, 0)),
        ],
        out_shape=[
            jax.ShapeDtypeStruct((n, d), jnp.float32),
            jax.ShapeDtypeStruct((n, d), jnp.float32),
        ],
    )
    e1, hs2 = mid(acc1, hs1, deg_col, b0r, W1)

    acc2 = edge_pass(hs2, ert, zeros2d)

    final = pl.pallas_call(
        _final_body,
        grid=grid,
        in_specs=[
            pl.BlockSpec((NC, br, d), lambda i: (0, i, 0)),
            pl.BlockSpec((br, d), lambda i: (i, 0)),
            pl.BlockSpec((br, 1), lambda i: (i, 0)),
            pl.BlockSpec((1, d), lambda i: (0, 0)),
        ],
        out_specs=pl.BlockSpec((br, d), lambda i: (i, 0)),
        out_shape=jax.ShapeDtypeStruct((n, d), jnp.float32),
    )
    out2 = final(acc2, hs2, deg_col, b1r)

    return (x, e1, out2)


# speedup vs baseline: 33.6377x; 1.0421x over previous
import functools

import jax
import jax.numpy as jnp
from jax import lax
from jax.experimental import pallas as pl
from jax.experimental.pallas import tpu as pltpu
from jax.experimental.pallas import tpu_sc as plsc

NC = 2
NS = 16
NW = NC * NS
K = 128
R = 3
RI = 5
F0 = 0.504


def _mesh():
    return plsc.VectorSubcoreMesh(
        core_axis_name="c", subcore_axis_name="s", num_cores=NC, num_subcores=NS
    )


def _chunk_range(ct):
    s0 = min(ct, max(0, int(round(ct * F0))))
    s1 = ct - s0
    return s0, s0 // NS, s0 % NS, s1 // NS, s1 % NS


def _worker_span(cid, sid, s0, q0, r0, q1, r1):
    nch = jnp.where(cid == 0, q0 + (sid < r0), q1 + (sid < r1))
    base = jnp.where(
        cid == 0,
        sid * q0 + jnp.minimum(sid, r0),
        s0 + sid * q1 + jnp.minimum(sid, r1),
    )
    return nch.astype(jnp.int32), base.astype(jnp.int32)


def _slab_copy(src, dst, sid, n):
    slab = ((n // NS) // 8) * 8
    last = n - (NS - 1) * slab

    @pl.when(sid < NS - 1)
    def _():
        pltpu.sync_copy(
            src.at[pl.ds(sid * slab, slab)], dst.at[pl.ds(sid * slab, slab)]
        )

    @pl.when(sid == NS - 1)
    def _():
        pltpu.sync_copy(
            src.at[pl.ds((NS - 1) * slab, last)],
            dst.at[pl.ds((NS - 1) * slab, last)],
        )


def _make_deg_kernel(nhist, ct):
    slab = nhist // NS
    s0, q0, r0, q1, r1 = _chunk_range(ct)

    def body(ert, out, idxd, ones_v, zbuf, deg_sh, sem_i, sem_s):
        cid = lax.axis_index("c")
        sid = lax.axis_index("s")
        nch, base = _worker_span(cid, sid, s0, q0, r0, q1, r1)
        for j in range(K // 16):
            ones_v[pl.ds(j * 16, 16)] = jnp.full((16,), 1.0, jnp.float32)
        for j in range(slab // 16):
            zbuf[pl.ds(j * 16, 16)] = jnp.zeros((16,), jnp.float32)
        pltpu.sync_copy(zbuf, deg_sh.at[pl.ds(sid * slab, slab)])

        def fetch(c, b):
            pltpu.async_copy(ert.at[base + c, 1], idxd.at[b], sem_i.at[b])

        def wait_fetch(c, b):
            pltpu.make_async_copy(
                ert.at[base + c, 1], idxd.at[b], sem_i.at[b]
            ).wait()

        def wait_scat(b):
            pltpu.make_async_copy(
                ones_v, deg_sh.at[idxd.at[b, 0]], sem_s.at[b]
            ).wait()

        for b in range(RI):
            @pl.when(b < nch)
            def _():
                fetch(b, b)
        plsc.subcore_barrier()

        def chunk(i, carry):
            bi = lax.rem(i, RI)
            bs = lax.rem(i, 2)
            wait_fetch(i, bi)
            pltpu.async_copy(
                ones_v, deg_sh.at[idxd.at[bi, 0]], sem_s.at[bs], add=True
            )

            @pl.when(i > 0)
            def _():
                wait_scat(lax.rem(i - 1, 2))

                @pl.when(i - 1 + RI < nch)
                def _():
                    fetch(i - 1 + RI, lax.rem(i - 1, RI))

            return carry

        lax.fori_loop(0, nch, chunk, 0)

        @pl.when(nch > 0)
        def _():
            wait_scat(lax.rem(nch - 1, 2))

        plsc.subcore_barrier()
        pltpu.sync_copy(
            deg_sh.at[pl.ds(sid * slab, slab)],
            out.at[cid].at[pl.ds(sid * slab, slab)],
        )

    return pl.kernel(
        body,
        out_type=jax.ShapeDtypeStruct((NC, nhist), jnp.float32),
        mesh=_mesh(),
        scratch_types=[
            pltpu.VMEM((RI, 1, K), jnp.int32),
            pltpu.VMEM((K,), jnp.float32),
            pltpu.VMEM((nhist // NS,), jnp.float32),
            pltpu.VMEM_SHARED((nhist,), jnp.float32),
            pltpu.SemaphoreType.DMA((RI,)),
            pltpu.SemaphoreType.DMA((2,)),
        ],
    )


def _make_edge_kernel(n, d, ct):
    s0, q0, r0, q1, r1 = _chunk_range(ct)

    def body(hs_hbm, ert, zeros_hbm, out, idxb, rows, acc,
             sem_i, sem_g, sem_s):
        cid = lax.axis_index("c")
        sid = lax.axis_index("s")
        nch, base = _worker_span(cid, sid, s0, q0, r0, q1, r1)

        @pl.when(cid == 0)
        def _():
            _slab_copy(hs_hbm, acc, sid, n)

        @pl.when(cid == 1)
        def _():
            _slab_copy(zeros_hbm, acc, sid, n)

        def fetch_idx(c, b):
            pltpu.async_copy(
                ert.at[base + c, 0], idxb.at[b, pl.ds(0, 1)], sem_i.at[b]
            )
            pltpu.async_copy(
                ert.at[base + c, 1], idxb.at[b, pl.ds(1, 1)], sem_i.at[b]
            )

        def wait_idx(c, b):
            pltpu.make_async_copy(
                ert.at[base + c, 0], idxb.at[b, pl.ds(0, 1)], sem_i.at[b]
            ).wait()
            pltpu.make_async_copy(
                ert.at[base + c, 1], idxb.at[b, pl.ds(1, 1)], sem_i.at[b]
            ).wait()

        def issue_gather(bi, b):
            pltpu.async_copy(
                hs_hbm.at[idxb.at[bi, 0]], rows.at[b], sem_g.at[b]
            )

        def wait_gather(bi, b):
            pltpu.make_async_copy(
                hs_hbm.at[idxb.at[bi, 0]], rows.at[b], sem_g.at[b]
            ).wait()

        def wait_scat(bi, b):
            pltpu.make_async_copy(
                rows.at[b], acc.at[idxb.at[bi, 1]], sem_s.at[b]
            ).wait()

        for b in range(RI):
            @pl.when(b < nch)
            def _():
                fetch_idx(b, b)
        for b in range(R):
            @pl.when(b < nch)
            def _():
                wait_idx(b, b)
                issue_gather(b, b)
        plsc.subcore_barrier()

        def chunk(i, carry):
            b = lax.rem(i, R)
            bi = lax.rem(i, RI)
            wait_gather(bi, b)
            pltpu.async_copy(
                rows.at[b], acc.at[idxb.at[bi, 1]], sem_s.at[b], add=True
            )

            @pl.when(i > 0)
            def _():
                ip = i - 1
                bp = lax.rem(ip, R)
                bip = lax.rem(ip, RI)
                wait_scat(bip, bp)

                @pl.when(ip + RI < nch)
                def _():
                    fetch_idx(ip + RI, bip)

                j = i + R - 1

                @pl.when(j < nch)
                def _():
                    bij = lax.rem(j, RI)
                    wait_idx(j, bij)
                    issue_gather(bij, bp)

            return carry

        lax.fori_loop(0, nch, chunk, 0)

        @pl.when(nch > 0)
        def _():
            wait_scat(lax.rem(nch - 1, RI), lax.rem(nch - 1, R))

        plsc.subcore_barrier()
        _slab_copy(acc, out.at[cid], sid, n)

    return pl.kernel(
        body,
        out_type=jax.ShapeDtypeStruct((NC, n, d), jnp.float32),
        mesh=_mesh(),
        scratch_types=[
            pltpu.VMEM((RI, 2, K), jnp.int32),
            pltpu.VMEM((R, K, d), jnp.float32),
            pltpu.VMEM_SHARED((n, d), jnp.float32),
            pltpu.SemaphoreType.DMA((RI,)),
            pltpu.SemaphoreType.DMA((R,)),
            pltpu.SemaphoreType.DMA((R,)),
        ],
    )


def _mm_scale_body(deg_ref, x_ref, w_ref, hs_ref):
    dinv = lax.rsqrt(deg_ref[...] + 1.0)
    h = jnp.dot(x_ref[...], w_ref[...], preferred_element_type=jnp.float32)
    hs_ref[...] = h * dinv


def _mid_body(acc_ref, deg_ref, b_ref, w_ref, e_ref, hs2_ref):
    dinv = lax.rsqrt(deg_ref[...] + 1.0)
    t = (acc_ref[0] + acc_ref[1]) * dinv + b_ref[...]
    e = jnp.maximum(t, 0.0)
    e_ref[...] = e
    h2 = jnp.dot(e, w_ref[...], preferred_element_type=jnp.float32)
    hs2_ref[...] = h2 * dinv


def _final_body(acc_ref, deg_ref, b_ref, out_ref):
    dinv = lax.rsqrt(deg_ref[...] + 1.0)
    out_ref[...] = (acc_ref[0] + acc_ref[1]) * dinv + b_ref[...]


def kernel(x, edge_index, W0, b0, W1, b1):
    n, d = x.shape
    e = edge_index.shape[1]
    assert e % K == 0 and n % 8 == 0
    ct = e // K
    nhist = ((n + 255) // 256) * 256

    ert = jnp.transpose(edge_index.reshape(2, ct, 1, K), (1, 0, 2, 3))
    zeros2d = jnp.zeros((n, d), jnp.float32)
    b0r = b0.reshape(1, d)
    b1r = b1.reshape(1, d)

    deg_part = _make_deg_kernel(nhist, ct)(ert)
    deg_col = (deg_part[0] + deg_part[1])[:n].reshape(n, 1)

    br = 1264
    grid = (-(-n // br),)
    mm_scale = pl.pallas_call(
        _mm_scale_body,
        grid=grid,
        in_specs=[
            pl.BlockSpec((br, 1), lambda i: (i, 0)),
            pl.BlockSpec((br, d), lambda i: (i, 0)),
            pl.BlockSpec((d, d), lambda i: (0, 0)),
        ],
        out_specs=pl.BlockSpec((br, d), lambda i: (i, 0)),
        out_shape=jax.ShapeDtypeStruct((n, d), jnp.float32),
    )
    hs1 = mm_scale(deg_col, x, W0)

    edge_pass = _make_edge_kernel(n, d, ct)
    acc1 = edge_pass(hs1, ert, zeros2d)

    mid = pl.pallas_call(
        _mid_body,
        grid=grid,
        in_specs=[
            pl.BlockSpec((NC, br, d), lambda i: (0, i, 0)),
            pl.BlockSpec((br, 1), lambda i: (i, 0)),
            pl.BlockSpec((1, d), lambda i: (0, 0)),
            pl.BlockSpec((d, d), lambda i: (0, 0)),
        ],
        out_specs=[
            pl.BlockSpec((br, d), lambda i: (i, 0)),
            pl.BlockSpec((br, d), lambda i: (i, 0)),
        ],
        out_shape=[
            jax.ShapeDtypeStruct((n, d), jnp.float32),
            jax.ShapeDtypeStruct((n, d), jnp.float32),
        ],
    )
    e1, hs2 = mid(acc1, deg_col, b0r, W1)

    acc2 = edge_pass(hs2, ert, zeros2d)

    final = pl.pallas_call(
        _final_body,
        grid=grid,
        in_specs=[
            pl.BlockSpec((NC, br, d), lambda i: (0, i, 0)),
            pl.BlockSpec((br, 1), lambda i: (i, 0)),
            pl.BlockSpec((1, d), lambda i: (0, 0)),
        ],
        out_specs=pl.BlockSpec((br, d), lambda i: (i, 0)),
        out_shape=jax.ShapeDtypeStruct((n, d), jnp.float32),
    )
    out2 = final(acc2, deg_col, b1r)

    return (x, e1, out2)


# speedup vs baseline: 33.6844x; 1.0014x over previous
import functools

import jax
import jax.numpy as jnp
from jax import lax
from jax.experimental import pallas as pl
from jax.experimental.pallas import tpu as pltpu
from jax.experimental.pallas import tpu_sc as plsc

NC = 2
NS = 16
NW = NC * NS
K = 128
R = 3
RI = 5
F0 = 0.504


def _mesh():
    return plsc.VectorSubcoreMesh(
        core_axis_name="c", subcore_axis_name="s", num_cores=NC, num_subcores=NS
    )


def _chunk_range(ct):
    s0 = min(ct, max(0, int(round(ct * F0))))
    s1 = ct - s0
    return s0, s0 // NS, s0 % NS, s1 // NS, s1 % NS


def _worker_span(cid, sid, s0, q0, r0, q1, r1):
    nch = jnp.where(cid == 0, q0 + (sid < r0), q1 + (sid < r1))
    base = jnp.where(
        cid == 0,
        sid * q0 + jnp.minimum(sid, r0),
        s0 + sid * q1 + jnp.minimum(sid, r1),
    )
    return nch.astype(jnp.int32), base.astype(jnp.int32)


def _slab_copy(src, dst, sid, n):
    slab = ((n // NS) // 8) * 8
    last = n - (NS - 1) * slab

    @pl.when(sid < NS - 1)
    def _():
        pltpu.sync_copy(
            src.at[pl.ds(sid * slab, slab)], dst.at[pl.ds(sid * slab, slab)]
        )

    @pl.when(sid == NS - 1)
    def _():
        pltpu.sync_copy(
            src.at[pl.ds((NS - 1) * slab, last)],
            dst.at[pl.ds((NS - 1) * slab, last)],
        )


def _make_deg_kernel(nhist, ct):
    slab = nhist // NS
    s0, q0, r0, q1, r1 = _chunk_range(ct)

    def body(ert, out, idxd, ones_v, zbuf, deg_sh, sem_i, sem_s):
        cid = lax.axis_index("c")
        sid = lax.axis_index("s")
        nch, base = _worker_span(cid, sid, s0, q0, r0, q1, r1)
        for j in range(K // 16):
            ones_v[pl.ds(j * 16, 16)] = jnp.full((16,), 1.0, jnp.float32)
        for j in range(slab // 16):
            zbuf[pl.ds(j * 16, 16)] = jnp.zeros((16,), jnp.float32)
        pltpu.sync_copy(zbuf, deg_sh.at[pl.ds(sid * slab, slab)])

        def fetch(c, b):
            pltpu.async_copy(ert.at[base + c, 1], idxd.at[b], sem_i.at[b])

        def wait_fetch(c, b):
            pltpu.make_async_copy(
                ert.at[base + c, 1], idxd.at[b], sem_i.at[b]
            ).wait()

        def wait_scat(b):
            pltpu.make_async_copy(
                ones_v, deg_sh.at[idxd.at[b, 0]], sem_s.at[b]
            ).wait()

        for b in range(RI):
            @pl.when(b < nch)
            def _():
                fetch(b, b)
        plsc.subcore_barrier()

        def chunk(i, carry):
            bi = lax.rem(i, RI)
            bs = lax.rem(i, 2)
            wait_fetch(i, bi)
            pltpu.async_copy(
                ones_v, deg_sh.at[idxd.at[bi, 0]], sem_s.at[bs], add=True
            )

            @pl.when(i > 0)
            def _():
                wait_scat(lax.rem(i - 1, 2))

                @pl.when(i - 1 + RI < nch)
                def _():
                    fetch(i - 1 + RI, lax.rem(i - 1, RI))

            return carry

        lax.fori_loop(0, nch, chunk, 0)

        @pl.when(nch > 0)
        def _():
            wait_scat(lax.rem(nch - 1, 2))

        plsc.subcore_barrier()
        pltpu.sync_copy(
            deg_sh.at[pl.ds(sid * slab, slab)],
            out.at[cid].at[pl.ds(sid * slab, slab)],
        )

    return pl.kernel(
        body,
        out_type=jax.ShapeDtypeStruct((NC, nhist), jnp.float32),
        mesh=_mesh(),
        scratch_types=[
            pltpu.VMEM((RI, 1, K), jnp.int32),
            pltpu.VMEM((K,), jnp.float32),
            pltpu.VMEM((nhist // NS,), jnp.float32),
            pltpu.VMEM_SHARED((nhist,), jnp.float32),
            pltpu.SemaphoreType.DMA((RI,)),
            pltpu.SemaphoreType.DMA((2,)),
        ],
    )


def _make_edge_kernel(n, d, ct):
    s0, q0, r0, q1, r1 = _chunk_range(ct)

    def body(hs_hbm, ert, zeros_hbm, out, idxb, rows, acc,
             sem_i, sem_g, sem_s):
        cid = lax.axis_index("c")
        sid = lax.axis_index("s")
        nch, base = _worker_span(cid, sid, s0, q0, r0, q1, r1)

        @pl.when(cid == 0)
        def _():
            _slab_copy(hs_hbm, acc, sid, n)

        @pl.when(cid == 1)
        def _():
            _slab_copy(zeros_hbm, acc, sid, n)

        def fetch_idx(c, b):
            pltpu.async_copy(ert.at[base + c], idxb.at[b], sem_i.at[b])

        def wait_idx(c, b):
            pltpu.make_async_copy(
                ert.at[base + c], idxb.at[b], sem_i.at[b]
            ).wait()

        def issue_gather(bi, b):
            pltpu.async_copy(
                hs_hbm.at[idxb.at[bi, 0, 0]], rows.at[b], sem_g.at[b]
            )

        def wait_gather(bi, b):
            pltpu.make_async_copy(
                hs_hbm.at[idxb.at[bi, 0, 0]], rows.at[b], sem_g.at[b]
            ).wait()

        def wait_scat(bi, b):
            pltpu.make_async_copy(
                rows.at[b], acc.at[idxb.at[bi, 1, 0]], sem_s.at[b]
            ).wait()

        for b in range(RI):
            @pl.when(b < nch)
            def _():
                fetch_idx(b, b)
        for b in range(R):
            @pl.when(b < nch)
            def _():
                wait_idx(b, b)
                issue_gather(b, b)
        plsc.subcore_barrier()

        def chunk(i, carry):
            b = lax.rem(i, R)
            bi = lax.rem(i, RI)
            wait_gather(bi, b)
            pltpu.async_copy(
                rows.at[b], acc.at[idxb.at[bi, 1, 0]], sem_s.at[b], add=True
            )

            @pl.when(i > 0)
            def _():
                ip = i - 1
                bp = lax.rem(ip, R)
                bip = lax.rem(ip, RI)
                wait_scat(bip, bp)

                @pl.when(ip + RI < nch)
                def _():
                    fetch_idx(ip + RI, bip)

                j = i + R - 1

                @pl.when(j < nch)
                def _():
                    bij = lax.rem(j, RI)
                    wait_idx(j, bij)
                    issue_gather(bij, bp)

            return carry

        lax.fori_loop(0, nch, chunk, 0)

        @pl.when(nch > 0)
        def _():
            wait_scat(lax.rem(nch - 1, RI), lax.rem(nch - 1, R))

        plsc.subcore_barrier()
        _slab_copy(acc, out.at[cid], sid, n)

    return pl.kernel(
        body,
        out_type=jax.ShapeDtypeStruct((NC, n, d), jnp.float32),
        mesh=_mesh(),
        scratch_types=[
            pltpu.VMEM((RI, 2, 1, K), jnp.int32),
            pltpu.VMEM((R, K, d), jnp.float32),
            pltpu.VMEM_SHARED((n, d), jnp.float32),
            pltpu.SemaphoreType.DMA((RI,)),
            pltpu.SemaphoreType.DMA((R,)),
            pltpu.SemaphoreType.DMA((R,)),
        ],
    )


def _mm_scale_body(deg_ref, x_ref, w_ref, hs_ref):
    dinv = lax.rsqrt(deg_ref[...] + 1.0)
    h = jnp.dot(x_ref[...], w_ref[...], preferred_element_type=jnp.float32)
    hs_ref[...] = h * dinv


def _mid_body(acc_ref, deg_ref, b_ref, w_ref, e_ref, hs2_ref):
    dinv = lax.rsqrt(deg_ref[...] + 1.0)
    t = (acc_ref[0] + acc_ref[1]) * dinv + b_ref[...]
    e = jnp.maximum(t, 0.0)
    e_ref[...] = e
    h2 = jnp.dot(e, w_ref[...], preferred_element_type=jnp.float32)
    hs2_ref[...] = h2 * dinv


def _final_body(acc_ref, deg_ref, b_ref, out_ref):
    dinv = lax.rsqrt(deg_ref[...] + 1.0)
    out_ref[...] = (acc_ref[0] + acc_ref[1]) * dinv + b_ref[...]


def kernel(x, edge_index, W0, b0, W1, b1):
    n, d = x.shape
    e = edge_index.shape[1]
    assert e % K == 0 and n % 8 == 0
    ct = e // K
    nhist = ((n + 255) // 256) * 256

    ert = jnp.transpose(edge_index.reshape(2, ct, 1, K), (1, 0, 2, 3))
    zeros2d = jnp.zeros((n, d), jnp.float32)
    b0r = b0.reshape(1, d)
    b1r = b1.reshape(1, d)

    deg_part = _make_deg_kernel(nhist, ct)(ert)
    deg_col = (deg_part[0] + deg_part[1])[:n].reshape(n, 1)

    br = 1264
    grid = (-(-n // br),)
    mm_scale = pl.pallas_call(
        _mm_scale_body,
        grid=grid,
        in_specs=[
            pl.BlockSpec((br, 1), lambda i: (i, 0)),
            pl.BlockSpec((br, d), lambda i: (i, 0)),
            pl.BlockSpec((d, d), lambda i: (0, 0)),
        ],
        out_specs=pl.BlockSpec((br, d), lambda i: (i, 0)),
        out_shape=jax.ShapeDtypeStruct((n, d), jnp.float32),
    )
    hs1 = mm_scale(deg_col, x, W0)

    edge_pass = _make_edge_kernel(n, d, ct)
    acc1 = edge_pass(hs1, ert, zeros2d)

    mid = pl.pallas_call(
        _mid_body,
        grid=grid,
        in_specs=[
            pl.BlockSpec((NC, br, d), lambda i: (0, i, 0)),
            pl.BlockSpec((br, 1), lambda i: (i, 0)),
            pl.BlockSpec((1, d), lambda i: (0, 0)),
            pl.BlockSpec((d, d), lambda i: (0, 0)),
        ],
        out_specs=[
            pl.BlockSpec((br, d), lambda i: (i, 0)),
            pl.BlockSpec((br, d), lambda i: (i, 0)),
        ],
        out_shape=[
            jax.ShapeDtypeStruct((n, d), jnp.float32),
            jax.ShapeDtypeStruct((n, d), jnp.float32),
        ],
    )
    e1, hs2 = mid(acc1, deg_col, b0r, W1)

    acc2 = edge_pass(hs2, ert, zeros2d)

    final = pl.pallas_call(
        _final_body,
        grid=grid,
        in_specs=[
            pl.BlockSpec((NC, br, d), lambda i: (0, i, 0)),
            pl.BlockSpec((br, 1), lambda i: (i, 0)),
            pl.BlockSpec((1, d), lambda i: (0, 0)),
        ],
        out_specs=pl.BlockSpec((br, d), lambda i: (i, 0)),
        out_shape=jax.ShapeDtypeStruct((n, d), jnp.float32),
    )
    out2 = final(acc2, deg_col, b1r)

    return (x, e1, out2)


# speedup vs baseline: 33.9871x; 1.0090x over previous
import functools

import jax
import jax.numpy as jnp
from jax import lax
from jax.experimental import pallas as pl
from jax.experimental.pallas import tpu as pltpu
from jax.experimental.pallas import tpu_sc as plsc

NC = 2
NS = 16
NW = NC * NS
K = 128
R = 3
RI = 5
F0 = 0.504


def _mesh():
    return plsc.VectorSubcoreMesh(
        core_axis_name="c", subcore_axis_name="s", num_cores=NC, num_subcores=NS
    )


def _chunk_range(ct):
    s0 = min(ct, max(0, int(round(ct * F0))))
    s1 = ct - s0
    return s0, s0 // NS, s0 % NS, s1 // NS, s1 % NS


def _worker_span(cid, sid, s0, q0, r0, q1, r1):
    nch = jnp.where(cid == 0, q0 + (sid < r0), q1 + (sid < r1))
    base = jnp.where(
        cid == 0,
        sid * q0 + jnp.minimum(sid, r0),
        s0 + sid * q1 + jnp.minimum(sid, r1),
    )
    return nch.astype(jnp.int32), base.astype(jnp.int32)


def _slab_copy(src, dst, sid, n):
    slab = ((n // NS) // 8) * 8
    last = n - (NS - 1) * slab

    @pl.when(sid < NS - 1)
    def _():
        pltpu.sync_copy(
            src.at[pl.ds(sid * slab, slab)], dst.at[pl.ds(sid * slab, slab)]
        )

    @pl.when(sid == NS - 1)
    def _():
        pltpu.sync_copy(
            src.at[pl.ds((NS - 1) * slab, last)],
            dst.at[pl.ds((NS - 1) * slab, last)],
        )


def _make_deg_kernel(nhist, ct):
    slab = nhist // NS
    s0, q0, r0, q1, r1 = _chunk_range(ct)

    def body(ert, out, idxd, ones_v, zbuf, deg_sh, sem_i, sem_s):
        cid = lax.axis_index("c")
        sid = lax.axis_index("s")
        nch, base = _worker_span(cid, sid, s0, q0, r0, q1, r1)
        for j in range(K // 16):
            ones_v[pl.ds(j * 16, 16)] = jnp.full((16,), 1.0, jnp.float32)
        for j in range(slab // 16):
            zbuf[pl.ds(j * 16, 16)] = jnp.zeros((16,), jnp.float32)
        pltpu.sync_copy(zbuf, deg_sh.at[pl.ds(sid * slab, slab)])

        def fetch(c, b):
            pltpu.async_copy(ert.at[base + c, 1], idxd.at[b], sem_i.at[b])

        def wait_fetch(c, b):
            pltpu.make_async_copy(
                ert.at[base + c, 1], idxd.at[b], sem_i.at[b]
            ).wait()

        def wait_scat(b):
            pltpu.make_async_copy(
                ones_v, deg_sh.at[idxd.at[b, 0]], sem_s.at[b]
            ).wait()

        for b in range(RI):
            @pl.when(b < nch)
            def _():
                fetch(b, b)
        plsc.subcore_barrier()

        def chunk(i, carry):
            bi = lax.rem(i, RI)
            bs = lax.rem(i, 2)
            wait_fetch(i, bi)
            pltpu.async_copy(
                ones_v, deg_sh.at[idxd.at[bi, 0]], sem_s.at[bs], add=True
            )

            @pl.when(i > 0)
            def _():
                wait_scat(lax.rem(i - 1, 2))

                @pl.when(i - 1 + RI < nch)
                def _():
                    fetch(i - 1 + RI, lax.rem(i - 1, RI))

            return carry

        lax.fori_loop(0, nch, chunk, 0)

        @pl.when(nch > 0)
        def _():
            wait_scat(lax.rem(nch - 1, 2))

        plsc.subcore_barrier()
        pltpu.sync_copy(
            deg_sh.at[pl.ds(sid * slab, slab)],
            out.at[cid].at[pl.ds(sid * slab, slab)],
        )

    return pl.kernel(
        body,
        out_type=jax.ShapeDtypeStruct((NC, nhist), jnp.float32),
        mesh=_mesh(),
        scratch_types=[
            pltpu.VMEM((RI, 1, K), jnp.int32),
            pltpu.VMEM((K,), jnp.float32),
            pltpu.VMEM((nhist // NS,), jnp.float32),
            pltpu.VMEM_SHARED((nhist,), jnp.float32),
            pltpu.SemaphoreType.DMA((RI,)),
            pltpu.SemaphoreType.DMA((2,)),
        ],
    )


def _make_edge_kernel(n, d, ct):
    s0, q0, r0, q1, r1 = _chunk_range(ct)

    def body(hs_hbm, ert, zeros_hbm, out, idxb, rows, acc,
             sem_i, sem_g, sem_s):
        cid = lax.axis_index("c")
        sid = lax.axis_index("s")
        nch, base = _worker_span(cid, sid, s0, q0, r0, q1, r1)

        @pl.when(cid == 0)
        def _():
            _slab_copy(hs_hbm, acc, sid, n)

        @pl.when(cid == 1)
        def _():
            _slab_copy(zeros_hbm, acc, sid, n)

        def fetch_idx(c, b):
            pltpu.async_copy(ert.at[base + c], idxb.at[b], sem_i.at[b])

        def wait_idx(c, b):
            pltpu.make_async_copy(
                ert.at[base + c], idxb.at[b], sem_i.at[b]
            ).wait()

        def issue_gather(bi, b):
            pltpu.async_copy(
                hs_hbm.at[idxb.at[bi, 0, 0]], rows.at[b], sem_g.at[b]
            )

        def wait_gather(bi, b):
            pltpu.make_async_copy(
                hs_hbm.at[idxb.at[bi, 0, 0]], rows.at[b], sem_g.at[b]
            ).wait()

        def wait_scat(bi, b):
            pltpu.make_async_copy(
                rows.at[b], acc.at[idxb.at[bi, 1, 0]], sem_s.at[b]
            ).wait()

        for b in range(RI):
            @pl.when(b < nch)
            def _():
                fetch_idx(b, b)
        for b in range(R):
            @pl.when(b < nch)
            def _():
                wait_idx(b, b)
                issue_gather(b, b)
        plsc.subcore_barrier()

        def chunk(i, carry):
            b = lax.rem(i, R)
            bi = lax.rem(i, RI)
            wait_gather(bi, b)
            pltpu.async_copy(
                rows.at[b], acc.at[idxb.at[bi, 1, 0]], sem_s.at[b], add=True
            )

            @pl.when(i > 0)
            def _():
                ip = i - 1
                bp = lax.rem(ip, R)
                bip = lax.rem(ip, RI)
                wait_scat(bip, bp)

                @pl.when(ip + RI < nch)
                def _():
                    fetch_idx(ip + RI, bip)

                j = i + R - 1

                @pl.when(j < nch)
                def _():
                    bij = lax.rem(j, RI)
                    wait_idx(j, bij)
                    issue_gather(bij, bp)

            return carry

        lax.fori_loop(0, nch, chunk, 0)

        @pl.when(nch > 0)
        def _():
            wait_scat(lax.rem(nch - 1, RI), lax.rem(nch - 1, R))

        plsc.subcore_barrier()
        _slab_copy(acc, out.at[cid], sid, n)

    return pl.kernel(
        body,
        out_type=jax.ShapeDtypeStruct((NC, n, d), jnp.float32),
        mesh=_mesh(),
        scratch_types=[
            pltpu.VMEM((RI, 2, 1, K), jnp.int32),
            pltpu.VMEM((R, K, d), jnp.float32),
            pltpu.VMEM_SHARED((n, d), jnp.float32),
            pltpu.SemaphoreType.DMA((RI,)),
            pltpu.SemaphoreType.DMA((R,)),
            pltpu.SemaphoreType.DMA((R,)),
        ],
    )


def _mm_scale_body(deg_ref, x_ref, w_ref, hs_ref):
    dinv = lax.rsqrt(deg_ref[...] + 1.0)
    h = jnp.dot(x_ref[...], w_ref[...], preferred_element_type=jnp.float32)
    hs_ref[...] = h * dinv


def _mid_body(acc_ref, deg_ref, b_ref, w_ref, e_ref, hs2_ref):
    dinv = lax.rsqrt(deg_ref[...] + 1.0)
    t = (acc_ref[0] + acc_ref[1]) * dinv + b_ref[...]
    e = jnp.maximum(t, 0.0)
    e_ref[...] = e
    h2 = jnp.dot(e, w_ref[...], preferred_element_type=jnp.float32)
    hs2_ref[...] = h2 * dinv


def _final_body(acc_ref, deg_ref, b_ref, out_ref):
    dinv = lax.rsqrt(deg_ref[...] + 1.0)
    out_ref[...] = (acc_ref[0] + acc_ref[1]) * dinv + b_ref[...]


def kernel(x, edge_index, W0, b0, W1, b1):
    n, d = x.shape
    e = edge_index.shape[1]
    assert e % K == 0 and n % 8 == 0
    ct = e // K
    nhist = ((n + 255) // 256) * 256

    ert = jnp.transpose(edge_index.reshape(2, ct, 1, K), (1, 0, 2, 3))
    zeros2d = jnp.zeros((n, d), jnp.float32)
    b0r = b0.reshape(1, d)
    b1r = b1.reshape(1, d)

    deg_part = _make_deg_kernel(nhist, ct)(ert)
    deg_col = (deg_part[0] + deg_part[1])[:n].reshape(n, 1)

    br = 2048
    grid = (-(-n // br),)
    mm_scale = pl.pallas_call(
        _mm_scale_body,
        grid=grid,
        in_specs=[
            pl.BlockSpec((br, 1), lambda i: (i, 0)),
            pl.BlockSpec((br, d), lambda i: (i, 0)),
            pl.BlockSpec((d, d), lambda i: (0, 0)),
        ],
        out_specs=pl.BlockSpec((br, d), lambda i: (i, 0)),
        out_shape=jax.ShapeDtypeStruct((n, d), jnp.float32),
    )
    hs1 = mm_scale(deg_col, x, W0)

    edge_pass = _make_edge_kernel(n, d, ct)
    acc1 = edge_pass(hs1, ert, zeros2d)

    mid = pl.pallas_call(
        _mid_body,
        grid=grid,
        in_specs=[
            pl.BlockSpec((NC, br, d), lambda i: (0, i, 0)),
            pl.BlockSpec((br, 1), lambda i: (i, 0)),
            pl.BlockSpec((1, d), lambda i: (0, 0)),
            pl.BlockSpec((d, d), lambda i: (0, 0)),
        ],
        out_specs=[
            pl.BlockSpec((br, d), lambda i: (i, 0)),
            pl.BlockSpec((br, d), lambda i: (i, 0)),
        ],
        out_shape=[
            jax.ShapeDtypeStruct((n, d), jnp.float32),
            jax.ShapeDtypeStruct((n, d), jnp.float32),
        ],
    )
    e1, hs2 = mid(acc1, deg_col, b0r, W1)

    acc2 = edge_pass(hs2, ert, zeros2d)

    final = pl.pallas_call(
        _final_body,
        grid=grid,
        in_specs=[
            pl.BlockSpec((NC, br, d), lambda i: (0, i, 0)),
            pl.BlockSpec((br, 1), lambda i: (i, 0)),
            pl.BlockSpec((1, d), lambda i: (0, 0)),
        ],
        out_specs=pl.BlockSpec((br, d), lambda i: (i, 0)),
        out_shape=jax.ShapeDtypeStruct((n, d), jnp.float32),
    )
    out2 = final(acc2, deg_col, b1r)

    return (x, e1, out2)


# speedup vs baseline: 34.0261x; 1.0011x over previous
import jax
import jax.numpy as jnp
from jax import lax
from jax.experimental import pallas as pl
from jax.experimental.pallas import tpu as pltpu
from jax.experimental.pallas import tpu_sc as plsc

NC = 2
NS = 16
NW = NC * NS
K = 128
R = 3
RI = 5
F0 = 0.504


def _mesh():
    return plsc.VectorSubcoreMesh(
        core_axis_name="c", subcore_axis_name="s", num_cores=NC, num_subcores=NS
    )


def _chunk_range(ct):
    s0 = min(ct, max(0, int(round(ct * F0))))
    s1 = ct - s0
    return s0, s0 // NS, s0 % NS, s1 // NS, s1 % NS


def _worker_span(cid, sid, s0, q0, r0, q1, r1):
    nch = jnp.where(cid == 0, q0 + (sid < r0), q1 + (sid < r1))
    base = jnp.where(
        cid == 0,
        sid * q0 + jnp.minimum(sid, r0),
        s0 + sid * q1 + jnp.minimum(sid, r1),
    )
    return nch.astype(jnp.int32), base.astype(jnp.int32)


def _slab_copy(src, dst, sid, n):
    slab = ((n // NS) // 8) * 8
    last = n - (NS - 1) * slab

    @pl.when(sid < NS - 1)
    def _():
        pltpu.sync_copy(
            src.at[pl.ds(sid * slab, slab)], dst.at[pl.ds(sid * slab, slab)]
        )

    @pl.when(sid == NS - 1)
    def _():
        pltpu.sync_copy(
            src.at[pl.ds((NS - 1) * slab, last)],
            dst.at[pl.ds((NS - 1) * slab, last)],
        )


def _make_deg_kernel(nhist, ct):
    slab = nhist // NS
    s0, q0, r0, q1, r1 = _chunk_range(ct)

    def body(ert, out, idxd, ones_v, zbuf, deg_sh, sem_i, sem_s):
        cid = lax.axis_index("c")
        sid = lax.axis_index("s")
        nch, base = _worker_span(cid, sid, s0, q0, r0, q1, r1)
        for j in range(K // 16):
            ones_v[pl.ds(j * 16, 16)] = jnp.full((16,), 1.0, jnp.float32)
        for j in range(slab // 16):
            zbuf[pl.ds(j * 16, 16)] = jnp.zeros((16,), jnp.float32)
        pltpu.sync_copy(zbuf, deg_sh.at[pl.ds(sid * slab, slab)])

        def fetch(c, b):
            pltpu.async_copy(ert.at[base + c, 1], idxd.at[b], sem_i.at[b])

        def wait_fetch(c, b):
            pltpu.make_async_copy(
                ert.at[base + c, 1], idxd.at[b], sem_i.at[b]
            ).wait()

        def wait_scat(b):
            pltpu.make_async_copy(
                ones_v, deg_sh.at[idxd.at[b, 0]], sem_s.at[b]
            ).wait()

        for b in range(RI):
            @pl.when(b < nch)
            def _():
                fetch(b, b)
        plsc.subcore_barrier()

        def chunk(i, carry):
            bi = lax.rem(i, RI)
            bs = lax.rem(i, 2)
            wait_fetch(i, bi)
            pltpu.async_copy(
                ones_v, deg_sh.at[idxd.at[bi, 0]], sem_s.at[bs], add=True
            )

            @pl.when(i > 0)
            def _():
                wait_scat(lax.rem(i - 1, 2))

                @pl.when(i - 1 + RI < nch)
                def _():
                    fetch(i - 1 + RI, lax.rem(i - 1, RI))

            return carry

        lax.fori_loop(0, nch, chunk, 0)

        @pl.when(nch > 0)
        def _():
            wait_scat(lax.rem(nch - 1, 2))

        plsc.subcore_barrier()
        pltpu.sync_copy(
            deg_sh.at[pl.ds(sid * slab, slab)],
            out.at[cid].at[pl.ds(sid * slab, slab)],
        )

    return pl.kernel(
        body,
        out_type=jax.ShapeDtypeStruct((NC, nhist), jnp.float32),
        mesh=_mesh(),
        scratch_types=[
            pltpu.VMEM((RI, 1, K), jnp.int32),
            pltpu.VMEM((K,), jnp.float32),
            pltpu.VMEM((nhist // NS,), jnp.float32),
            pltpu.VMEM_SHARED((nhist,), jnp.float32),
            pltpu.SemaphoreType.DMA((RI,)),
            pltpu.SemaphoreType.DMA((2,)),
        ],
    )


def _make_edge_kernel(n, d, ct):
    s0, q0, r0, q1, r1 = _chunk_range(ct)

    def body(hs_hbm, ert, zeros_hbm, out, idxb, rows, acc,
             sem_i, sem_g, sem_s):
        cid = lax.axis_index("c")
        sid = lax.axis_index("s")
        nch, base = _worker_span(cid, sid, s0, q0, r0, q1, r1)

        @pl.when(cid == 0)
        def _():
            _slab_copy(hs_hbm, acc, sid, n)

        @pl.when(cid == 1)
        def _():
            _slab_copy(zeros_hbm, acc, sid, n)

        def fetch_idx(c, b):
            pltpu.async_copy(ert.at[base + c], idxb.at[b], sem_i.at[b])

        def wait_idx(c, b):
            pltpu.make_async_copy(
                ert.at[base + c], idxb.at[b], sem_i.at[b]
            ).wait()

        def issue_gather(bi, b):
            pltpu.async_copy(
                hs_hbm.at[idxb.at[bi, 0, 0]], rows.at[b], sem_g.at[b]
            )

        def wait_gather(bi, b):
            pltpu.make_async_copy(
                hs_hbm.at[idxb.at[bi, 0, 0]], rows.at[b], sem_g.at[b]
            ).wait()

        def wait_scat(bi, b):
            pltpu.make_async_copy(
                rows.at[b], acc.at[idxb.at[bi, 1, 0]], sem_s.at[b]
            ).wait()

        for b in range(RI):
            @pl.when(b < nch)
            def _():
                fetch_idx(b, b)
        for b in range(R):
            @pl.when(b < nch)
            def _():
                wait_idx(b, b)
                issue_gather(b, b)
        plsc.subcore_barrier()

        def chunk(i, carry):
            b = lax.rem(i, R)
            bi = lax.rem(i, RI)
            wait_gather(bi, b)
            pltpu.async_copy(
                rows.at[b], acc.at[idxb.at[bi, 1, 0]], sem_s.at[b], add=True
            )

            @pl.when(i > 0)
            def _():
                ip = i - 1
                bp = lax.rem(ip, R)
                bip = lax.rem(ip, RI)
                wait_scat(bip, bp)

                @pl.when(ip + RI < nch)
                def _():
                    fetch_idx(ip + RI, bip)

                j = i + R - 1

                @pl.when(j < nch)
                def _():
                    bij = lax.rem(j, RI)
                    wait_idx(j, bij)
                    issue_gather(bij, bp)

            return carry

        lax.fori_loop(0, nch, chunk, 0)

        @pl.when(nch > 0)
        def _():
            wait_scat(lax.rem(nch - 1, RI), lax.rem(nch - 1, R))

        plsc.subcore_barrier()
        _slab_copy(acc, out.at[cid], sid, n)

    return pl.kernel(
        body,
        out_type=jax.ShapeDtypeStruct((NC, n, d), jnp.float32),
        mesh=_mesh(),
        scratch_types=[
            pltpu.VMEM((RI, 2, 1, K), jnp.int32),
            pltpu.VMEM((R, K, d), jnp.float32),
            pltpu.VMEM_SHARED((n, d), jnp.float32),
            pltpu.SemaphoreType.DMA((RI,)),
            pltpu.SemaphoreType.DMA((R,)),
            pltpu.SemaphoreType.DMA((R,)),
        ],
    )


def _mm_scale_body(deg_ref, x_ref, w_ref, hs_ref):
    dinv = lax.rsqrt(deg_ref[...] + 1.0)
    h = jnp.dot(x_ref[...], w_ref[...], preferred_element_type=jnp.float32)
    hs_ref[...] = h * dinv


def _mid_body(acc_ref, deg_ref, b_ref, w_ref, e_ref, hs2_ref):
    dinv = lax.rsqrt(deg_ref[...] + 1.0)
    t = (acc_ref[0] + acc_ref[1]) * dinv + b_ref[...]
    e = jnp.maximum(t, 0.0)
    e_ref[...] = e
    h2 = jnp.dot(e, w_ref[...], preferred_element_type=jnp.float32)
    hs2_ref[...] = h2 * dinv


def _final_body(acc_ref, deg_ref, b_ref, out_ref):
    dinv = lax.rsqrt(deg_ref[...] + 1.0)
    out_ref[...] = (acc_ref[0] + acc_ref[1]) * dinv + b_ref[...]


def kernel(x, edge_index, W0, b0, W1, b1):
    n, d = x.shape
    e = edge_index.shape[1]
    assert e % K == 0 and n % 8 == 0
    ct = e // K
    nhist = ((n + 255) // 256) * 256

    ert = jnp.transpose(edge_index.reshape(2, ct, 1, K), (1, 0, 2, 3))
    zeros2d = jnp.zeros((n, d), jnp.float32)
    b0r = b0.reshape(1, d)
    b1r = b1.reshape(1, d)

    deg_part = _make_deg_kernel(nhist, ct)(ert)
    deg_col = (deg_part[0] + deg_part[1])[:n].reshape(n, 1)

    br = 2048
    grid = (-(-n // br),)
    mm_scale = pl.pallas_call(
        _mm_scale_body,
        grid=grid,
        in_specs=[
            pl.BlockSpec((br, 1), lambda i: (i, 0)),
            pl.BlockSpec((br, d), lambda i: (i, 0)),
            pl.BlockSpec((d, d), lambda i: (0, 0)),
        ],
        out_specs=pl.BlockSpec((br, d), lambda i: (i, 0)),
        out_shape=jax.ShapeDtypeStruct((n, d), jnp.float32),
    )
    hs1 = mm_scale(deg_col, x, W0)

    edge_pass = _make_edge_kernel(n, d, ct)
    acc1 = edge_pass(hs1, ert, zeros2d)

    mid = pl.pallas_call(
        _mid_body,
        grid=grid,
        in_specs=[
            pl.BlockSpec((NC, br, d), lambda i: (0, i, 0)),
            pl.BlockSpec((br, 1), lambda i: (i, 0)),
            pl.BlockSpec((1, d), lambda i: (0, 0)),
            pl.BlockSpec((d, d), lambda i: (0, 0)),
        ],
        out_specs=[
            pl.BlockSpec((br, d), lambda i: (i, 0)),
            pl.BlockSpec((br, d), lambda i: (i, 0)),
        ],
        out_shape=[
            jax.ShapeDtypeStruct((n, d), jnp.float32),
            jax.ShapeDtypeStruct((n, d), jnp.float32),
        ],
    )
    e1, hs2 = mid(acc1, deg_col, b0r, W1)

    acc2 = edge_pass(hs2, ert, zeros2d)

    final = pl.pallas_call(
        _final_body,
        grid=grid,
        in_specs=[
            pl.BlockSpec((NC, br, d), lambda i: (0, i, 0)),
            pl.BlockSpec((br, 1), lambda i: (i, 0)),
            pl.BlockSpec((1, d), lambda i: (0, 0)),
        ],
        out_specs=pl.BlockSpec((br, d), lambda i: (i, 0)),
        out_shape=jax.ShapeDtypeStruct((n, d), jnp.float32),
    )
    out2 = final(acc2, deg_col, b1r)

    return (x, e1, out2)
